# baseline (speedup 1.0000x reference)
import jax
import jax.numpy as jnp
from jax.experimental import pallas as pl
from jax.experimental.pallas import tpu as pltpu
from jax.experimental.pallas import tpu_sc as plsc

S, D, H, E, FF, TOPK = 2048, 768, 12, 8, 1024, 2
DH = D // H
BT = 256
NT = S // BT
NEG = -1e30

NA = TOPK * S
AR, AC = 32, 128
BME = 256
P = 6144
NB = P // BME
WSC = 128


def _qkv_kernel(x_ref, ln1_ref, wq_ref, wk_ref, wv_ref, cos_ref, sin_ref,
                q_ref, k_ref, v_ref):
    x = x_ref[...]
    var = jnp.mean(jnp.square(x), axis=-1, keepdims=True)
    h = (x * jax.lax.rsqrt(var + 1e-6) * ln1_ref[...]).astype(jnp.bfloat16)
    q = jnp.dot(h, wq_ref[...], preferred_element_type=jnp.float32)
    k = jnp.dot(h, wk_ref[...], preferred_element_type=jnp.float32)
    v = jnp.dot(h, wv_ref[...], preferred_element_type=jnp.float32)
    cos = cos_ref[...]
    sin = sin_ref[...]
    col = jax.lax.broadcasted_iota(jnp.int32, (BT, D), 1)
    first_half = (col % DH) < (DH // 2)

    def rope(u):
        rot = jnp.where(first_half,
                        -pltpu.roll(u, D - DH // 2, 1),
                        pltpu.roll(u, DH // 2, 1))
        return u * cos + rot * sin

    q_ref[...] = (rope(q) * (1.0 / (DH ** 0.5))).astype(jnp.bfloat16)
    k_ref[...] = rope(k).astype(jnp.bfloat16)
    v_ref[...] = v.astype(jnp.bfloat16)


BQ = 512
NQ = S // BQ


def _attn_kernel(q_ref, k_ref, v_ref, o_ref):
    qi = pl.program_id(1)
    q = q_ref[0]

    def step(kb, carry, masked):
        m, l, acc = carry
        k = k_ref[0, pl.ds(kb * BQ, BQ), :]
        s = jax.lax.dot_general(q, k, (((1,), (1,)), ((), ())),
                                preferred_element_type=jnp.float32)
        if masked:
            rows = jax.lax.broadcasted_iota(jnp.int32, (BQ, BQ), 0)
            cols = jax.lax.broadcasted_iota(jnp.int32, (BQ, BQ), 1)
            s = jnp.where(rows >= cols, s, NEG)
        m_new = jnp.maximum(m, jnp.max(s, axis=1, keepdims=True))
        alpha = jnp.exp(m - m_new)
        p = jnp.exp(s - m_new)
        l = l * alpha + jnp.sum(p, axis=1, keepdims=True)
        v = v_ref[0, pl.ds(kb * BQ, BQ), :]
        pv = jnp.dot(p.astype(jnp.bfloat16), v,
                     preferred_element_type=jnp.float32)
        acc = acc * alpha + pv
        return m_new, l, acc

    m0 = jnp.full((BQ, 1), NEG, jnp.float32)
    l0 = jnp.zeros((BQ, 1), jnp.float32)
    a0 = jnp.zeros((BQ, DH), jnp.float32)
    carry = jax.lax.fori_loop(0, qi, lambda kb, c: step(kb, c, False),
                              (m0, l0, a0))
    m, l, acc = step(qi, carry, True)
    o_ref[0] = (acc / l).astype(jnp.bfloat16)


def _post_attn_kernel(ctx_ref, wo_ref, dec_ref, ln2_ref, wr_ref,
                      h2_ref, x2_ref, ti_ref, tw_ref):
    ctx = ctx_ref[...]
    h2 = dec_ref[...] + jnp.dot(ctx, wo_ref[...],
                                preferred_element_type=jnp.float32)
    h2_ref[...] = h2
    var = jnp.mean(jnp.square(h2), axis=-1, keepdims=True)
    x2 = h2 * jax.lax.rsqrt(var + 1e-6) * ln2_ref[...]
    x2_ref[...] = x2.astype(jnp.bfloat16)
    logits = jax.lax.dot_general(x2, wr_ref[...], (((1,), (0,)), ((), ())),
                                 precision=jax.lax.Precision.HIGHEST,
                                 preferred_element_type=jnp.float32)
    col = jax.lax.broadcasted_iota(jnp.int32, (BT, E), 1)
    m1 = jnp.max(logits, axis=1, keepdims=True)
    i1 = jnp.min(jnp.where(logits == m1, col, E), axis=1, keepdims=True)
    masked = jnp.where(col == i1, NEG, logits)
    m2 = jnp.max(masked, axis=1, keepdims=True)
    i2 = jnp.min(jnp.where(masked == m2, col, E), axis=1, keepdims=True)
    w1 = 1.0 / (1.0 + jnp.exp(m2 - m1))
    w2 = 1.0 - w1
    two = jax.lax.broadcasted_iota(jnp.int32, (BT, TOPK), 1)
    ti_ref[...] = jnp.where(two == 0, i1, i2)
    tw_ref[...] = jnp.where(two == 0, w1, w2)


def _dispatch_kernel(ej_ref, pos_ref, be_ref, act_ref):
    ej = ej_ref[...]
    lane = jax.lax.broadcasted_iota(jnp.int32, (AR, AC), 1)
    srow = jax.lax.broadcasted_iota(jnp.int32, (AR, 1), 0)
    nb_iota = jax.lax.broadcasted_iota(jnp.int32, (1, NB), 1)
    pos = jnp.zeros((AR, AC), jnp.int32)
    be = jnp.zeros((1, NB), jnp.int32)
    off = jnp.int32(0)
    for e in range(E):
        m = (ej == e).astype(jnp.int32)
        pr = m
        for sh in (1, 2, 4, 8, 16, 32, 64):
            pr = pr + jnp.where(lane >= sh, pltpu.roll(pr, sh, 1), 0)
        rowtot = jnp.sum(m, axis=1, keepdims=True)
        rp = rowtot
        for sh in (1, 2, 4, 8, 16):
            rp = rp + jnp.where(srow >= sh, pltpu.roll(rp, sh, 0), 0)
        rank = (pr - m) + (rp - rowtot)
        cnt = jnp.sum(m)
        padded = ((cnt + BME - 1) // BME) * BME
        pos = jnp.where(ej == e, off + rank, pos)
        start_b = off // BME
        nblk = padded // BME
        be = jnp.where((nb_iota >= start_b) & (nb_iota < start_b + nblk),
                       e, be)
        off = off + padded
    pos_ref[...] = pos
    be_ref[...] = be
    act_ref[...] = (nb_iota < off // BME).astype(jnp.int32)


def _gmm_kernel(be_ref, act_ref, xg_ref, wg_ref, wu_ref, wd_ref, yg_ref):
    @pl.when(act_ref[pl.program_id(0)] != 0)
    def _():
        x = xg_ref[...]
        g = jnp.dot(x, wg_ref[0], preferred_element_type=jnp.float32)
        u = jnp.dot(x, wu_ref[0], preferred_element_type=jnp.float32)
        act = (g * jax.nn.sigmoid(g) * u).astype(jnp.bfloat16)
        yg_ref[...] = jnp.dot(act, wd_ref[0],
                              preferred_element_type=jnp.float32
                              ).astype(jnp.bfloat16)


def _final_kernel(h2_ref, a_ref, b_ref, tw_ref, out_ref):
    tw = tw_ref[...]
    w0 = tw[:, 0:1]
    w1 = tw[:, 1:2]
    out_ref[...] = (h2_ref[...]
                    + w0 * a_ref[...].astype(jnp.float32)
                    + w1 * b_ref[...].astype(jnp.float32))


def _sc_mesh():
    return plsc.VectorSubcoreMesh(core_axis_name="c", subcore_axis_name="s")


D2 = D // 2


def _sc_scatter(x2i, p0, p1):
    @pl.kernel(out_type=jax.ShapeDtypeStruct((P, D2), jnp.int32),
               mesh=_sc_mesh())
    def scat(x2_hbm, p0_hbm, p1_hbm, xg_hbm):
        def body(x_vmem, i0_vmem, i1_vmem):
            pltpu.sync_copy(x_vmem, xg_hbm.at[i0_vmem.at[0]])
            pltpu.sync_copy(x_vmem, xg_hbm.at[i1_vmem.at[0]])

        pltpu.emit_pipeline(
            body,
            grid=(S // WSC,),
            in_specs=[pl.BlockSpec((WSC, D2), lambda i: (i, 0)),
                      pl.BlockSpec((1, WSC), lambda i: (0, i)),
                      pl.BlockSpec((1, WSC), lambda i: (0, i))],
            out_specs=[],
            core_axis_name=("c", "s"),
            dimension_semantics=(pltpu.PARALLEL,),
        )(x2_hbm, p0_hbm, p1_hbm)

    return scat(x2i, p0, p1)


def _sc_gather(ygi, p0, p1):
    @pl.kernel(out_type=[jax.ShapeDtypeStruct((S, D2), jnp.int32)] * 2,
               mesh=_sc_mesh())
    def gath(yg_hbm, p0_hbm, p1_hbm, a_hbm, b_hbm):
        def body_a(i0_vmem, a_vmem):
            pltpu.sync_copy(yg_hbm.at[i0_vmem.at[0]], a_vmem)

        def body_b(i1_vmem, b_vmem):
            pltpu.sync_copy(yg_hbm.at[i1_vmem.at[0]], b_vmem)

        pltpu.emit_pipeline(
            body_a,
            grid=(S // WSC,),
            in_specs=[pl.BlockSpec((1, WSC), lambda i: (0, i))],
            out_specs=[pl.BlockSpec((WSC, D2), lambda i: (i, 0))],
            core_axis_name=("c", "s"),
            dimension_semantics=(pltpu.PARALLEL,),
        )(p0_hbm, a_hbm)
        pltpu.emit_pipeline(
            body_b,
            grid=(S // WSC,),
            in_specs=[pl.BlockSpec((1, WSC), lambda i: (0, i))],
            out_specs=[pl.BlockSpec((WSC, D2), lambda i: (i, 0))],
            core_axis_name=("c", "s"),
            dimension_semantics=(pltpu.PARALLEL,),
        )(p1_hbm, b_hbm)

    return gath(ygi, p0, p1)


def kernel(dec_inp, ln1_w, ln2_w, Wq, Wk, Wv, Wo, Wrouter, Wgate, Wup, Wdown):
    b, s, d = dec_inp.shape
    x = dec_inp.reshape(s, d)
    ln1 = ln1_w.reshape(1, d)
    ln2 = ln2_w.reshape(1, d)
    wq = Wq.astype(jnp.bfloat16)
    wk = Wk.astype(jnp.bfloat16)
    wv = Wv.astype(jnp.bfloat16)
    wo = Wo.astype(jnp.bfloat16)
    wg = Wgate.astype(jnp.bfloat16)
    wu = Wup.astype(jnp.bfloat16)
    wd = Wdown.astype(jnp.bfloat16)

    inv_freq = 1.0 / (10000.0 ** (jnp.arange(0, DH, 2, dtype=jnp.float32) / DH))
    t = jnp.arange(s, dtype=jnp.float32)
    freqs = jnp.outer(t, inv_freq)
    emb = jnp.concatenate([freqs, freqs], axis=-1)
    cos = jnp.tile(jnp.cos(emb), (1, H))
    sin = jnp.tile(jnp.sin(emb), (1, H))

    bf = jnp.bfloat16
    q, k, v = pl.pallas_call(
        _qkv_kernel,
        grid=(NT,),
        in_specs=[
            pl.BlockSpec((BT, D), lambda i: (i, 0)),
            pl.BlockSpec((1, D), lambda i: (0, 0)),
            pl.BlockSpec((D, D), lambda i: (0, 0)),
            pl.BlockSpec((D, D), lambda i: (0, 0)),
            pl.BlockSpec((D, D), lambda i: (0, 0)),
            pl.BlockSpec((BT, D), lambda i: (i, 0)),
            pl.BlockSpec((BT, D), lambda i: (i, 0)),
        ],
        out_specs=[
            pl.BlockSpec((BT, D), lambda i: (i, 0)),
            pl.BlockSpec((BT, D), lambda i: (i, 0)),
            pl.BlockSpec((BT, D), lambda i: (i, 0)),
        ],
        out_shape=[jax.ShapeDtypeStruct((s, d), bf)] * 3,
        compiler_params=pltpu.CompilerParams(
            dimension_semantics=("parallel",)),
    )(x, ln1, wq, wk, wv, cos, sin)

    qh = q.reshape(s, H, DH).transpose(1, 0, 2)
    kh = k.reshape(s, H, DH).transpose(1, 0, 2)
    vh = v.reshape(s, H, DH).transpose(1, 0, 2)
    ctx_h = pl.pallas_call(
        _attn_kernel,
        grid=(H, NQ),
        in_specs=[
            pl.BlockSpec((1, BQ, DH), lambda h, i: (h, i, 0)),
            pl.BlockSpec((1, S, DH), lambda h, i: (h, 0, 0)),
            pl.BlockSpec((1, S, DH), lambda h, i: (h, 0, 0)),
        ],
        out_specs=pl.BlockSpec((1, BQ, DH), lambda h, i: (h, i, 0)),
        out_shape=jax.ShapeDtypeStruct((H, s, DH), bf),
        compiler_params=pltpu.CompilerParams(
            dimension_semantics=("parallel", "arbitrary")),
    )(qh, kh, vh)
    ctx = ctx_h.transpose(1, 0, 2).reshape(s, d)

    h2, x2, ti, tw = pl.pallas_call(
        _post_attn_kernel,
        grid=(NT,),
        in_specs=[
            pl.BlockSpec((BT, D), lambda i: (i, 0)),
            pl.BlockSpec((D, D), lambda i: (0, 0)),
            pl.BlockSpec((BT, D), lambda i: (i, 0)),
            pl.BlockSpec((1, D), lambda i: (0, 0)),
            pl.BlockSpec((D, E), lambda i: (0, 0)),
        ],
        out_specs=[
            pl.BlockSpec((BT, D), lambda i: (i, 0)),
            pl.BlockSpec((BT, D), lambda i: (i, 0)),
            pl.BlockSpec((BT, TOPK), lambda i: (i, 0)),
            pl.BlockSpec((BT, TOPK), lambda i: (i, 0)),
        ],
        out_shape=[
            jax.ShapeDtypeStruct((s, d), jnp.float32),
            jax.ShapeDtypeStruct((s, d), bf),
            jax.ShapeDtypeStruct((s, TOPK), jnp.int32),
            jax.ShapeDtypeStruct((s, TOPK), jnp.float32),
        ],
        compiler_params=pltpu.CompilerParams(
            dimension_semantics=("parallel",)),
    )(ctx, wo, x, ln2, Wrouter)

    ej = ti.reshape(AR, AC)
    pos, be, act = pl.pallas_call(
        _dispatch_kernel,
        grid=(1,),
        in_specs=[pl.BlockSpec((AR, AC), lambda i: (0, 0))],
        out_specs=[
            pl.BlockSpec((AR, AC), lambda i: (0, 0)),
            pl.BlockSpec((1, NB), lambda i: (0, 0)),
            pl.BlockSpec((1, NB), lambda i: (0, 0)),
        ],
        out_shape=[
            jax.ShapeDtypeStruct((AR, AC), jnp.int32),
            jax.ShapeDtypeStruct((1, NB), jnp.int32),
            jax.ShapeDtypeStruct((1, NB), jnp.int32),
        ],
    )(ej)
    pos_flat = pos.reshape(NA)
    p0 = pos_flat[0::2].reshape(1, S)
    p1 = pos_flat[1::2].reshape(1, S)

    x2i = jax.lax.bitcast_convert_type(x2.reshape(s, D2, 2), jnp.int32)
    xgi = jnp.concatenate([x2i, x2i, x2i], axis=0)[:P]
    xg = jax.lax.bitcast_convert_type(xgi, jnp.bfloat16).reshape(P, D)

    yg = pl.pallas_call(
        _gmm_kernel,
        grid_spec=pltpu.PrefetchScalarGridSpec(
            num_scalar_prefetch=2,
            grid=(NB,),
            in_specs=[
                pl.BlockSpec((BME, D), lambda i, be_, act_: (i, 0)),
                pl.BlockSpec((1, D, FF),
                             lambda i, be_, act_: (be_[i], 0, 0)),
                pl.BlockSpec((1, D, FF),
                             lambda i, be_, act_: (be_[i], 0, 0)),
                pl.BlockSpec((1, FF, D),
                             lambda i, be_, act_: (be_[i], 0, 0)),
            ],
            out_specs=pl.BlockSpec((BME, D), lambda i, be_, act_: (i, 0)),
        ),
        out_shape=jax.ShapeDtypeStruct((P, D), bf),
        compiler_params=pltpu.CompilerParams(
            dimension_semantics=("arbitrary",)),
    )(be.reshape(NB), act.reshape(NB), xg, wg, wu, wd)

    ygi = jax.lax.bitcast_convert_type(yg.reshape(P, D2, 2), jnp.int32)
    ai, bi = ygi[:S], ygi[S:2*S]
    _ = (p0, p1)
    a_rows = jax.lax.bitcast_convert_type(ai, jnp.bfloat16).reshape(s, d)
    b_rows = jax.lax.bitcast_convert_type(bi, jnp.bfloat16).reshape(s, d)

    out = pl.pallas_call(
        _final_kernel,
        grid=(NT,),
        in_specs=[
            pl.BlockSpec((BT, D), lambda i: (i, 0)),
            pl.BlockSpec((BT, D), lambda i: (i, 0)),
            pl.BlockSpec((BT, D), lambda i: (i, 0)),
            pl.BlockSpec((BT, TOPK), lambda i: (i, 0)),
        ],
        out_specs=pl.BlockSpec((BT, D), lambda i: (i, 0)),
        out_shape=jax.ShapeDtypeStruct((s, d), jnp.float32),
        compiler_params=pltpu.CompilerParams(
            dimension_semantics=("parallel",)),
    )(h2, a_rows, b_rows, tw)

    return out.reshape(b, s, d)


# speedup vs baseline: 1.7762x; 1.7762x over previous
import jax
import jax.numpy as jnp
from jax.experimental import pallas as pl
from jax.experimental.pallas import tpu as pltpu
from jax.experimental.pallas import tpu_sc as plsc

S, D, H, E, FF, TOPK = 2048, 768, 12, 8, 1024, 2
DH = D // H
BT = 256
NT = S // BT
NEG = -1e30

NA = TOPK * S
AR, AC = 32, 128
BME = 256
P = 6144
NB = P // BME
WSC = 128


def _qkv_kernel(x_ref, ln1_ref, wq_ref, wk_ref, wv_ref, cos_ref, sin_ref,
                q_ref, k_ref, v_ref):
    x = x_ref[...]
    var = jnp.mean(jnp.square(x), axis=-1, keepdims=True)
    h = (x * jax.lax.rsqrt(var + 1e-6) * ln1_ref[...]).astype(jnp.bfloat16)
    q = jnp.dot(h, wq_ref[...], preferred_element_type=jnp.float32)
    k = jnp.dot(h, wk_ref[...], preferred_element_type=jnp.float32)
    v = jnp.dot(h, wv_ref[...], preferred_element_type=jnp.float32)
    cos = cos_ref[...]
    sin = sin_ref[...]
    col = jax.lax.broadcasted_iota(jnp.int32, (BT, D), 1)
    first_half = (col % DH) < (DH // 2)

    def rope(u):
        rot = jnp.where(first_half,
                        -pltpu.roll(u, D - DH // 2, 1),
                        pltpu.roll(u, DH // 2, 1))
        return u * cos + rot * sin

    q_ref[...] = (rope(q) * (1.0 / (DH ** 0.5))).astype(jnp.bfloat16)
    k_ref[...] = rope(k).astype(jnp.bfloat16)
    v_ref[...] = v.astype(jnp.bfloat16)


BQ = 512
NQ = S // BQ


def _attn_kernel(q_ref, k_ref, v_ref, o_ref):
    qi = pl.program_id(1)
    q = q_ref[0]

    def step(kb, carry, masked):
        m, l, acc = carry
        k = k_ref[0, pl.ds(kb * BQ, BQ), :]
        s = jax.lax.dot_general(q, k, (((1,), (1,)), ((), ())),
                                preferred_element_type=jnp.float32)
        if masked:
            rows = jax.lax.broadcasted_iota(jnp.int32, (BQ, BQ), 0)
            cols = jax.lax.broadcasted_iota(jnp.int32, (BQ, BQ), 1)
            s = jnp.where(rows >= cols, s, NEG)
        m_new = jnp.maximum(m, jnp.max(s, axis=1, keepdims=True))
        alpha = jnp.exp(m - m_new)
        p = jnp.exp(s - m_new)
        l = l * alpha + jnp.sum(p, axis=1, keepdims=True)
        v = v_ref[0, pl.ds(kb * BQ, BQ), :]
        pv = jnp.dot(p.astype(jnp.bfloat16), v,
                     preferred_element_type=jnp.float32)
        acc = acc * alpha + pv
        return m_new, l, acc

    m0 = jnp.full((BQ, 1), NEG, jnp.float32)
    l0 = jnp.zeros((BQ, 1), jnp.float32)
    a0 = jnp.zeros((BQ, DH), jnp.float32)
    carry = jax.lax.fori_loop(0, qi, lambda kb, c: step(kb, c, False),
                              (m0, l0, a0))
    m, l, acc = step(qi, carry, True)
    o_ref[0] = (acc / l).astype(jnp.bfloat16)


def _post_attn_kernel(ctx_ref, wo_ref, dec_ref, ln2_ref, wr_ref,
                      h2_ref, x2_ref, ti_ref, tw_ref):
    ctx = ctx_ref[...]
    h2 = dec_ref[...] + jnp.dot(ctx, wo_ref[...],
                                preferred_element_type=jnp.float32)
    h2_ref[...] = h2
    var = jnp.mean(jnp.square(h2), axis=-1, keepdims=True)
    x2 = h2 * jax.lax.rsqrt(var + 1e-6) * ln2_ref[...]
    x2_ref[...] = _pack(x2.astype(jnp.bfloat16))
    logits = jax.lax.dot_general(x2, wr_ref[...], (((1,), (0,)), ((), ())),
                                 precision=jax.lax.Precision.HIGHEST,
                                 preferred_element_type=jnp.float32)
    col = jax.lax.broadcasted_iota(jnp.int32, (BT, E), 1)
    m1 = jnp.max(logits, axis=1, keepdims=True)
    i1 = jnp.min(jnp.where(logits == m1, col, E), axis=1, keepdims=True)
    masked = jnp.where(col == i1, NEG, logits)
    m2 = jnp.max(masked, axis=1, keepdims=True)
    i2 = jnp.min(jnp.where(masked == m2, col, E), axis=1, keepdims=True)
    w1 = 1.0 / (1.0 + jnp.exp(m2 - m1))
    w2 = 1.0 - w1
    two = jax.lax.broadcasted_iota(jnp.int32, (BT, TOPK), 1)
    ti_ref[...] = jnp.where(two == 0, i1, i2)
    tw_ref[...] = jnp.where(two == 0, w1, w2)


D2 = D // 2


def _pack(xb):
    lo = jax.lax.bitcast_convert_type(xb[:, :D2], jnp.uint16)
    hi = jax.lax.bitcast_convert_type(xb[:, D2:], jnp.uint16)
    u = (hi.astype(jnp.uint32) << 16) | lo.astype(jnp.uint32)
    return jax.lax.bitcast_convert_type(u, jnp.int32)


def _unpack(p):
    u = jax.lax.bitcast_convert_type(p, jnp.uint32)
    lo = jax.lax.bitcast_convert_type((u & 0xffff).astype(jnp.uint16),
                                      jnp.bfloat16)
    hi = jax.lax.bitcast_convert_type((u >> 16).astype(jnp.uint16),
                                      jnp.bfloat16)
    return jnp.concatenate([lo, hi], axis=1)


def _dispatch_kernel(ej_ref, pos_ref, be_ref, act_ref):
    ej = ej_ref[...]
    lane = jax.lax.broadcasted_iota(jnp.int32, (AR, AC), 1)
    srow = jax.lax.broadcasted_iota(jnp.int32, (AR, 1), 0)
    nb_iota = jax.lax.broadcasted_iota(jnp.int32, (1, NB), 1)
    pos = jnp.zeros((AR, AC), jnp.int32)
    be = jnp.zeros((1, NB), jnp.int32)
    off = jnp.int32(0)
    for e in range(E):
        m = (ej == e).astype(jnp.int32)
        pr = m
        for sh in (1, 2, 4, 8, 16, 32, 64):
            pr = pr + jnp.where(lane >= sh, pltpu.roll(pr, sh, 1), 0)
        rowtot = jnp.sum(m, axis=1, keepdims=True)
        rp = rowtot
        for sh in (1, 2, 4, 8, 16):
            rp = rp + jnp.where(srow >= sh, pltpu.roll(rp, sh, 0), 0)
        rank = (pr - m) + (rp - rowtot)
        cnt = jnp.sum(m)
        padded = ((cnt + BME - 1) // BME) * BME
        pos = jnp.where(ej == e, off + rank, pos)
        start_b = off // BME
        nblk = padded // BME
        be = jnp.where((nb_iota >= start_b) & (nb_iota < start_b + nblk),
                       e, be)
        off = off + padded
    pos_ref[...] = pos
    be_ref[...] = be
    act_ref[...] = (nb_iota < off // BME).astype(jnp.int32)


def _gmm_kernel(be_ref, act_ref, xg_ref, wg_ref, wu_ref, wd_ref, yg_ref):
    @pl.when(act_ref[pl.program_id(0)] != 0)
    def _():
        x = _unpack(xg_ref[...])
        g = jnp.dot(x, wg_ref[0], preferred_element_type=jnp.float32)
        u = jnp.dot(x, wu_ref[0], preferred_element_type=jnp.float32)
        act = (g * jax.nn.sigmoid(g) * u).astype(jnp.bfloat16)
        eo = jnp.dot(act, wd_ref[0], preferred_element_type=jnp.float32)
        yg_ref[...] = _pack(eo.astype(jnp.bfloat16))


def _final_kernel(h2_ref, a_ref, b_ref, tw_ref, out_ref):
    tw = tw_ref[...]
    w0 = tw[:, 0:1]
    w1 = tw[:, 1:2]
    out_ref[...] = (h2_ref[...]
                    + w0 * _unpack(a_ref[...]).astype(jnp.float32)
                    + w1 * _unpack(b_ref[...]).astype(jnp.float32))


def _sc_mesh():
    return plsc.VectorSubcoreMesh(core_axis_name="c", subcore_axis_name="s")


def _sc_scatter(x2i, p0, p1):
    @pl.kernel(out_type=jax.ShapeDtypeStruct((P, D2), jnp.int32),
               mesh=_sc_mesh())
    def scat(x2_hbm, p0_hbm, p1_hbm, xg_hbm):
        def body(x_vmem, i0_vmem, i1_vmem):
            pltpu.sync_copy(x_vmem, xg_hbm.at[i0_vmem.at[0]])
            pltpu.sync_copy(x_vmem, xg_hbm.at[i1_vmem.at[0]])

        pltpu.emit_pipeline(
            body,
            grid=(S // WSC,),
            in_specs=[pl.BlockSpec((WSC, D2), lambda i: (i, 0)),
                      pl.BlockSpec((1, WSC), lambda i: (0, i)),
                      pl.BlockSpec((1, WSC), lambda i: (0, i))],
            out_specs=[],
            core_axis_name=("c", "s"),
            dimension_semantics=(pltpu.PARALLEL,),
        )(x2_hbm, p0_hbm, p1_hbm)

    return scat(x2i, p0, p1)


def _sc_gather(ygi, p0, p1):
    @pl.kernel(out_type=[jax.ShapeDtypeStruct((S, D2), jnp.int32)] * 2,
               mesh=_sc_mesh())
    def gath(yg_hbm, p0_hbm, p1_hbm, a_hbm, b_hbm):
        def body_a(i0_vmem, a_vmem):
            pltpu.sync_copy(yg_hbm.at[i0_vmem.at[0]], a_vmem)

        def body_b(i1_vmem, b_vmem):
            pltpu.sync_copy(yg_hbm.at[i1_vmem.at[0]], b_vmem)

        pltpu.emit_pipeline(
            body_a,
            grid=(S // WSC,),
            in_specs=[pl.BlockSpec((1, WSC), lambda i: (0, i))],
            out_specs=[pl.BlockSpec((WSC, D2), lambda i: (i, 0))],
            core_axis_name=("c", "s"),
            dimension_semantics=(pltpu.PARALLEL,),
        )(p0_hbm, a_hbm)
        pltpu.emit_pipeline(
            body_b,
            grid=(S // WSC,),
            in_specs=[pl.BlockSpec((1, WSC), lambda i: (0, i))],
            out_specs=[pl.BlockSpec((WSC, D2), lambda i: (i, 0))],
            core_axis_name=("c", "s"),
            dimension_semantics=(pltpu.PARALLEL,),
        )(p1_hbm, b_hbm)

    return gath(ygi, p0, p1)


def kernel(dec_inp, ln1_w, ln2_w, Wq, Wk, Wv, Wo, Wrouter, Wgate, Wup, Wdown):
    b, s, d = dec_inp.shape
    x = dec_inp.reshape(s, d)
    ln1 = ln1_w.reshape(1, d)
    ln2 = ln2_w.reshape(1, d)
    wq = Wq.astype(jnp.bfloat16)
    wk = Wk.astype(jnp.bfloat16)
    wv = Wv.astype(jnp.bfloat16)
    wo = Wo.astype(jnp.bfloat16)
    wg = Wgate.astype(jnp.bfloat16)
    wu = Wup.astype(jnp.bfloat16)
    wd = Wdown.astype(jnp.bfloat16)

    inv_freq = 1.0 / (10000.0 ** (jnp.arange(0, DH, 2, dtype=jnp.float32) / DH))
    t = jnp.arange(s, dtype=jnp.float32)
    freqs = jnp.outer(t, inv_freq)
    emb = jnp.concatenate([freqs, freqs], axis=-1)
    cos = jnp.tile(jnp.cos(emb), (1, H))
    sin = jnp.tile(jnp.sin(emb), (1, H))

    bf = jnp.bfloat16
    q, k, v = pl.pallas_call(
        _qkv_kernel,
        grid=(NT,),
        in_specs=[
            pl.BlockSpec((BT, D), lambda i: (i, 0)),
            pl.BlockSpec((1, D), lambda i: (0, 0)),
            pl.BlockSpec((D, D), lambda i: (0, 0)),
            pl.BlockSpec((D, D), lambda i: (0, 0)),
            pl.BlockSpec((D, D), lambda i: (0, 0)),
            pl.BlockSpec((BT, D), lambda i: (i, 0)),
            pl.BlockSpec((BT, D), lambda i: (i, 0)),
        ],
        out_specs=[
            pl.BlockSpec((BT, D), lambda i: (i, 0)),
            pl.BlockSpec((BT, D), lambda i: (i, 0)),
            pl.BlockSpec((BT, D), lambda i: (i, 0)),
        ],
        out_shape=[jax.ShapeDtypeStruct((s, d), bf)] * 3,
        compiler_params=pltpu.CompilerParams(
            dimension_semantics=("parallel",)),
    )(x, ln1, wq, wk, wv, cos, sin)

    qh = q.reshape(s, H, DH).transpose(1, 0, 2)
    kh = k.reshape(s, H, DH).transpose(1, 0, 2)
    vh = v.reshape(s, H, DH).transpose(1, 0, 2)
    ctx_h = pl.pallas_call(
        _attn_kernel,
        grid=(H, NQ),
        in_specs=[
            pl.BlockSpec((1, BQ, DH), lambda h, i: (h, i, 0)),
            pl.BlockSpec((1, S, DH), lambda h, i: (h, 0, 0)),
            pl.BlockSpec((1, S, DH), lambda h, i: (h, 0, 0)),
        ],
        out_specs=pl.BlockSpec((1, BQ, DH), lambda h, i: (h, i, 0)),
        out_shape=jax.ShapeDtypeStruct((H, s, DH), bf),
        compiler_params=pltpu.CompilerParams(
            dimension_semantics=("parallel", "arbitrary")),
    )(qh, kh, vh)
    ctx = ctx_h.transpose(1, 0, 2).reshape(s, d)

    h2, x2, ti, tw = pl.pallas_call(
        _post_attn_kernel,
        grid=(NT,),
        in_specs=[
            pl.BlockSpec((BT, D), lambda i: (i, 0)),
            pl.BlockSpec((D, D), lambda i: (0, 0)),
            pl.BlockSpec((BT, D), lambda i: (i, 0)),
            pl.BlockSpec((1, D), lambda i: (0, 0)),
            pl.BlockSpec((D, E), lambda i: (0, 0)),
        ],
        out_specs=[
            pl.BlockSpec((BT, D), lambda i: (i, 0)),
            pl.BlockSpec((BT, D2), lambda i: (i, 0)),
            pl.BlockSpec((BT, TOPK), lambda i: (i, 0)),
            pl.BlockSpec((BT, TOPK), lambda i: (i, 0)),
        ],
        out_shape=[
            jax.ShapeDtypeStruct((s, d), jnp.float32),
            jax.ShapeDtypeStruct((s, D2), jnp.int32),
            jax.ShapeDtypeStruct((s, TOPK), jnp.int32),
            jax.ShapeDtypeStruct((s, TOPK), jnp.float32),
        ],
        compiler_params=pltpu.CompilerParams(
            dimension_semantics=("parallel",)),
    )(ctx, wo, x, ln2, Wrouter)

    ej = ti.reshape(AR, AC)
    pos, be, act = pl.pallas_call(
        _dispatch_kernel,
        grid=(1,),
        in_specs=[pl.BlockSpec((AR, AC), lambda i: (0, 0))],
        out_specs=[
            pl.BlockSpec((AR, AC), lambda i: (0, 0)),
            pl.BlockSpec((1, NB), lambda i: (0, 0)),
            pl.BlockSpec((1, NB), lambda i: (0, 0)),
        ],
        out_shape=[
            jax.ShapeDtypeStruct((AR, AC), jnp.int32),
            jax.ShapeDtypeStruct((1, NB), jnp.int32),
            jax.ShapeDtypeStruct((1, NB), jnp.int32),
        ],
    )(ej)
    pos_flat = pos.reshape(NA)
    p0 = pos_flat[0::2].reshape(1, S)
    p1 = pos_flat[1::2].reshape(1, S)

    xg = _sc_scatter(x2, p0, p1)

    yg = pl.pallas_call(
        _gmm_kernel,
        grid_spec=pltpu.PrefetchScalarGridSpec(
            num_scalar_prefetch=2,
            grid=(NB,),
            in_specs=[
                pl.BlockSpec((BME, D2), lambda i, be_, act_: (i, 0)),
                pl.BlockSpec((1, D, FF),
                             lambda i, be_, act_: (be_[i], 0, 0)),
                pl.BlockSpec((1, D, FF),
                             lambda i, be_, act_: (be_[i], 0, 0)),
                pl.BlockSpec((1, FF, D),
                             lambda i, be_, act_: (be_[i], 0, 0)),
            ],
            out_specs=pl.BlockSpec((BME, D2), lambda i, be_, act_: (i, 0)),
        ),
        out_shape=jax.ShapeDtypeStruct((P, D2), jnp.int32),
        compiler_params=pltpu.CompilerParams(
            dimension_semantics=("arbitrary",)),
    )(be.reshape(NB), act.reshape(NB), xg, wg, wu, wd)

    a_rows, b_rows = _sc_gather(yg, p0, p1)

    out = pl.pallas_call(
        _final_kernel,
        grid=(NT,),
        in_specs=[
            pl.BlockSpec((BT, D), lambda i: (i, 0)),
            pl.BlockSpec((BT, D2), lambda i: (i, 0)),
            pl.BlockSpec((BT, D2), lambda i: (i, 0)),
            pl.BlockSpec((BT, TOPK), lambda i: (i, 0)),
        ],
        out_specs=pl.BlockSpec((BT, D), lambda i: (i, 0)),
        out_shape=jax.ShapeDtypeStruct((s, d), jnp.float32),
        compiler_params=pltpu.CompilerParams(
            dimension_semantics=("parallel",)),
    )(h2, a_rows, b_rows, tw)

    return out.reshape(b, s, d)


# speedup vs baseline: 2.0268x; 1.1411x over previous
import jax
import jax.numpy as jnp
from jax.experimental import pallas as pl
from jax.experimental.pallas import tpu as pltpu
from jax.experimental.pallas import tpu_sc as plsc

S, D, H, E, FF, TOPK = 2048, 768, 12, 8, 1024, 2
DH = D // H
BT = 256
NT = S // BT
NEG = -1e30

NA = TOPK * S
AR, AC = 32, 128
BME = 256
P = 6144
NB = P // BME
WSC = 128


def _qkv_kernel(x_ref, ln1_ref, wq_ref, wk_ref, wv_ref, cos_ref, sin_ref,
                q_ref, k_ref, v_ref):
    x = x_ref[...]
    var = jnp.mean(jnp.square(x), axis=-1, keepdims=True)
    h = (x * jax.lax.rsqrt(var + 1e-6) * ln1_ref[...]).astype(jnp.bfloat16)
    q = jnp.dot(h, wq_ref[...], preferred_element_type=jnp.float32)
    k = jnp.dot(h, wk_ref[...], preferred_element_type=jnp.float32)
    v = jnp.dot(h, wv_ref[...], preferred_element_type=jnp.float32)
    cos = cos_ref[...]
    sin = sin_ref[...]
    col = jax.lax.broadcasted_iota(jnp.int32, (BT, D), 1)
    first_half = (col % DH) < (DH // 2)

    def rope(u):
        rot = jnp.where(first_half,
                        -pltpu.roll(u, D - DH // 2, 1),
                        pltpu.roll(u, DH // 2, 1))
        return u * cos + rot * sin

    q_ref[...] = (rope(q) * (1.0 / (DH ** 0.5))).astype(jnp.bfloat16)
    k_ref[...] = rope(k).astype(jnp.bfloat16)
    v_ref[...] = v.astype(jnp.bfloat16)


BQ = 512
NQ = S // BQ


def _attn_kernel(q_ref, k_ref, v_ref, o_ref):
    qi = pl.program_id(1)
    qs = (q_ref[:, :DH], q_ref[:, DH:])

    def step(kb, carry, masked):
        kk = k_ref[pl.ds(kb * BQ, BQ), :]
        vv = v_ref[pl.ds(kb * BQ, BQ), :]
        if masked:
            rows = jax.lax.broadcasted_iota(jnp.int32, (BQ, BQ), 0)
            cols = jax.lax.broadcasted_iota(jnp.int32, (BQ, BQ), 1)
            vis = rows >= cols
        out = []
        for hh in range(2):
            m, l, acc = carry[hh]
            k = kk[:, hh * DH:(hh + 1) * DH]
            s = jax.lax.dot_general(qs[hh], k, (((1,), (1,)), ((), ())),
                                    preferred_element_type=jnp.float32)
            if masked:
                s = jnp.where(vis, s, NEG)
            m_new = jnp.maximum(m, jnp.max(s, axis=1, keepdims=True))
            alpha = jnp.exp(m - m_new)
            p = jnp.exp(s - m_new)
            l = l * alpha + jnp.sum(p, axis=1, keepdims=True)
            v = vv[:, hh * DH:(hh + 1) * DH]
            pv = jnp.dot(p.astype(jnp.bfloat16), v,
                         preferred_element_type=jnp.float32)
            acc = acc * alpha + pv
            out.append((m_new, l, acc))
        return tuple(out)

    init = tuple((jnp.full((BQ, 1), NEG, jnp.float32),
                  jnp.zeros((BQ, 1), jnp.float32),
                  jnp.zeros((BQ, DH), jnp.float32)) for _ in range(2))
    carry = jax.lax.fori_loop(0, qi, lambda kb, c: step(kb, c, False), init)
    res = step(qi, carry, True)
    o_ref[...] = jnp.concatenate(
        [(acc / l).astype(jnp.bfloat16) for (m, l, acc) in res], axis=1)


def _post_attn_kernel(ctx_ref, wo_ref, dec_ref, ln2_ref, wr_ref,
                      h2_ref, x2_ref, ti_ref, tw_ref):
    ctx = ctx_ref[...]
    h2 = dec_ref[...] + jnp.dot(ctx, wo_ref[...],
                                preferred_element_type=jnp.float32)
    h2_ref[...] = h2
    var = jnp.mean(jnp.square(h2), axis=-1, keepdims=True)
    x2 = h2 * jax.lax.rsqrt(var + 1e-6) * ln2_ref[...]
    x2_ref[...] = _pack(x2.astype(jnp.bfloat16))
    logits = jax.lax.dot_general(x2, wr_ref[...], (((1,), (0,)), ((), ())),
                                 precision=jax.lax.Precision.HIGHEST,
                                 preferred_element_type=jnp.float32)
    col = jax.lax.broadcasted_iota(jnp.int32, (BT, E), 1)
    m1 = jnp.max(logits, axis=1, keepdims=True)
    i1 = jnp.min(jnp.where(logits == m1, col, E), axis=1, keepdims=True)
    masked = jnp.where(col == i1, NEG, logits)
    m2 = jnp.max(masked, axis=1, keepdims=True)
    i2 = jnp.min(jnp.where(masked == m2, col, E), axis=1, keepdims=True)
    w1 = 1.0 / (1.0 + jnp.exp(m2 - m1))
    w2 = 1.0 - w1
    two = jax.lax.broadcasted_iota(jnp.int32, (BT, TOPK), 1)
    ti_ref[...] = jnp.where(two == 0, i1, i2)
    tw_ref[...] = jnp.where(two == 0, w1, w2)


D2 = D // 2


def _pack(xb):
    lo = jax.lax.bitcast_convert_type(xb[:, :D2], jnp.uint16)
    hi = jax.lax.bitcast_convert_type(xb[:, D2:], jnp.uint16)
    u = (hi.astype(jnp.uint32) << 16) | lo.astype(jnp.uint32)
    return jax.lax.bitcast_convert_type(u, jnp.int32)


def _unpack(p):
    u = jax.lax.bitcast_convert_type(p, jnp.uint32)
    lo = jax.lax.bitcast_convert_type((u & 0xffff).astype(jnp.uint16),
                                      jnp.bfloat16)
    hi = jax.lax.bitcast_convert_type((u >> 16).astype(jnp.uint16),
                                      jnp.bfloat16)
    return jnp.concatenate([lo, hi], axis=1)


def _dispatch_kernel(ej_ref, pos_ref, be_ref, act_ref):
    ej = ej_ref[...]
    lane = jax.lax.broadcasted_iota(jnp.int32, (AR, AC), 1)
    srow = jax.lax.broadcasted_iota(jnp.int32, (AR, 1), 0)
    nb_iota = jax.lax.broadcasted_iota(jnp.int32, (1, NB), 1)
    pos = jnp.zeros((AR, AC), jnp.int32)
    be = jnp.zeros((1, NB), jnp.int32)
    off = jnp.int32(0)
    for e in range(E):
        m = (ej == e).astype(jnp.int32)
        pr = m
        for sh in (1, 2, 4, 8, 16, 32, 64):
            pr = pr + jnp.where(lane >= sh, pltpu.roll(pr, sh, 1), 0)
        rowtot = jnp.sum(m, axis=1, keepdims=True)
        rp = rowtot
        for sh in (1, 2, 4, 8, 16):
            rp = rp + jnp.where(srow >= sh, pltpu.roll(rp, sh, 0), 0)
        rank = (pr - m) + (rp - rowtot)
        cnt = jnp.sum(m)
        padded = ((cnt + BME - 1) // BME) * BME
        pos = jnp.where(ej == e, off + rank, pos)
        start_b = off // BME
        nblk = padded // BME
        be = jnp.where((nb_iota >= start_b) & (nb_iota < start_b + nblk),
                       e, be)
        off = off + padded
    pos_ref[...] = pos
    be_ref[...] = be
    act_ref[...] = (nb_iota < off // BME).astype(jnp.int32)


def _gmm_kernel(be_ref, act_ref, xg_ref, wg_ref, wu_ref, wd_ref, yg_ref):
    @pl.when(act_ref[pl.program_id(0)] != 0)
    def _():
        x = _unpack(xg_ref[...])
        g = jnp.dot(x, wg_ref[0], preferred_element_type=jnp.float32)
        u = jnp.dot(x, wu_ref[0], preferred_element_type=jnp.float32)
        act = (g * jax.nn.sigmoid(g) * u).astype(jnp.bfloat16)
        eo = jnp.dot(act, wd_ref[0], preferred_element_type=jnp.float32)
        yg_ref[...] = _pack(eo.astype(jnp.bfloat16))


def _final_kernel(h2_ref, a_ref, b_ref, tw_ref, out_ref):
    tw = tw_ref[...]
    w0 = tw[:, 0:1]
    w1 = tw[:, 1:2]
    out_ref[...] = (h2_ref[...]
                    + w0 * _unpack(a_ref[...]).astype(jnp.float32)
                    + w1 * _unpack(b_ref[...]).astype(jnp.float32))


def _sc_mesh():
    return plsc.VectorSubcoreMesh(core_axis_name="c", subcore_axis_name="s")


def _sc_scatter(x2i, p0, p1):
    @pl.kernel(out_type=jax.ShapeDtypeStruct((P, D2), jnp.int32),
               mesh=_sc_mesh())
    def scat(x2_hbm, p0_hbm, p1_hbm, xg_hbm):
        def body(x_vmem, i0_vmem, i1_vmem):
            pltpu.sync_copy(x_vmem, xg_hbm.at[i0_vmem.at[0]])
            pltpu.sync_copy(x_vmem, xg_hbm.at[i1_vmem.at[0]])

        pltpu.emit_pipeline(
            body,
            grid=(S // WSC,),
            in_specs=[pl.BlockSpec((WSC, D2), lambda i: (i, 0)),
                      pl.BlockSpec((1, WSC), lambda i: (0, i)),
                      pl.BlockSpec((1, WSC), lambda i: (0, i))],
            out_specs=[],
            core_axis_name=("c", "s"),
            dimension_semantics=(pltpu.PARALLEL,),
        )(x2_hbm, p0_hbm, p1_hbm)

    return scat(x2i, p0, p1)


def _sc_gather(ygi, p0, p1):
    @pl.kernel(out_type=[jax.ShapeDtypeStruct((S, D2), jnp.int32)] * 2,
               mesh=_sc_mesh())
    def gath(yg_hbm, p0_hbm, p1_hbm, a_hbm, b_hbm):
        def body_a(i0_vmem, a_vmem):
            pltpu.sync_copy(yg_hbm.at[i0_vmem.at[0]], a_vmem)

        def body_b(i1_vmem, b_vmem):
            pltpu.sync_copy(yg_hbm.at[i1_vmem.at[0]], b_vmem)

        pltpu.emit_pipeline(
            body_a,
            grid=(S // WSC,),
            in_specs=[pl.BlockSpec((1, WSC), lambda i: (0, i))],
            out_specs=[pl.BlockSpec((WSC, D2), lambda i: (i, 0))],
            core_axis_name=("c", "s"),
            dimension_semantics=(pltpu.PARALLEL,),
        )(p0_hbm, a_hbm)
        pltpu.emit_pipeline(
            body_b,
            grid=(S // WSC,),
            in_specs=[pl.BlockSpec((1, WSC), lambda i: (0, i))],
            out_specs=[pl.BlockSpec((WSC, D2), lambda i: (i, 0))],
            core_axis_name=("c", "s"),
            dimension_semantics=(pltpu.PARALLEL,),
        )(p1_hbm, b_hbm)

    return gath(ygi, p0, p1)


def kernel(dec_inp, ln1_w, ln2_w, Wq, Wk, Wv, Wo, Wrouter, Wgate, Wup, Wdown):
    b, s, d = dec_inp.shape
    x = dec_inp.reshape(s, d)
    ln1 = ln1_w.reshape(1, d)
    ln2 = ln2_w.reshape(1, d)
    wq = Wq.astype(jnp.bfloat16)
    wk = Wk.astype(jnp.bfloat16)
    wv = Wv.astype(jnp.bfloat16)
    wo = Wo.astype(jnp.bfloat16)
    wg = Wgate.astype(jnp.bfloat16)
    wu = Wup.astype(jnp.bfloat16)
    wd = Wdown.astype(jnp.bfloat16)

    inv_freq = 1.0 / (10000.0 ** (jnp.arange(0, DH, 2, dtype=jnp.float32) / DH))
    t = jnp.arange(s, dtype=jnp.float32)
    freqs = jnp.outer(t, inv_freq)
    emb = jnp.concatenate([freqs, freqs], axis=-1)
    cos = jnp.tile(jnp.cos(emb), (1, H))
    sin = jnp.tile(jnp.sin(emb), (1, H))

    bf = jnp.bfloat16
    q, k, v = pl.pallas_call(
        _qkv_kernel,
        grid=(NT,),
        in_specs=[
            pl.BlockSpec((BT, D), lambda i: (i, 0)),
            pl.BlockSpec((1, D), lambda i: (0, 0)),
            pl.BlockSpec((D, D), lambda i: (0, 0)),
            pl.BlockSpec((D, D), lambda i: (0, 0)),
            pl.BlockSpec((D, D), lambda i: (0, 0)),
            pl.BlockSpec((BT, D), lambda i: (i, 0)),
            pl.BlockSpec((BT, D), lambda i: (i, 0)),
        ],
        out_specs=[
            pl.BlockSpec((BT, D), lambda i: (i, 0)),
            pl.BlockSpec((BT, D), lambda i: (i, 0)),
            pl.BlockSpec((BT, D), lambda i: (i, 0)),
        ],
        out_shape=[jax.ShapeDtypeStruct((s, d), bf)] * 3,
        compiler_params=pltpu.CompilerParams(
            dimension_semantics=("parallel",)),
    )(x, ln1, wq, wk, wv, cos, sin)

    ctx = pl.pallas_call(
        _attn_kernel,
        grid=(H // 2, NQ),
        in_specs=[
            pl.BlockSpec((BQ, 2 * DH), lambda h, i: (i, h)),
            pl.BlockSpec((S, 2 * DH), lambda h, i: (0, h)),
            pl.BlockSpec((S, 2 * DH), lambda h, i: (0, h)),
        ],
        out_specs=pl.BlockSpec((BQ, 2 * DH), lambda h, i: (i, h)),
        out_shape=jax.ShapeDtypeStruct((s, d), bf),
        compiler_params=pltpu.CompilerParams(
            dimension_semantics=("parallel", "arbitrary")),
    )(q, k, v)

    h2, x2, ti, tw = pl.pallas_call(
        _post_attn_kernel,
        grid=(NT,),
        in_specs=[
            pl.BlockSpec((BT, D), lambda i: (i, 0)),
            pl.BlockSpec((D, D), lambda i: (0, 0)),
            pl.BlockSpec((BT, D), lambda i: (i, 0)),
            pl.BlockSpec((1, D), lambda i: (0, 0)),
            pl.BlockSpec((D, E), lambda i: (0, 0)),
        ],
        out_specs=[
            pl.BlockSpec((BT, D), lambda i: (i, 0)),
            pl.BlockSpec((BT, D2), lambda i: (i, 0)),
            pl.BlockSpec((BT, TOPK), lambda i: (i, 0)),
            pl.BlockSpec((BT, TOPK), lambda i: (i, 0)),
        ],
        out_shape=[
            jax.ShapeDtypeStruct((s, d), jnp.float32),
            jax.ShapeDtypeStruct((s, D2), jnp.int32),
            jax.ShapeDtypeStruct((s, TOPK), jnp.int32),
            jax.ShapeDtypeStruct((s, TOPK), jnp.float32),
        ],
        compiler_params=pltpu.CompilerParams(
            dimension_semantics=("parallel",)),
    )(ctx, wo, x, ln2, Wrouter)

    ej = ti.reshape(AR, AC)
    pos, be, act = pl.pallas_call(
        _dispatch_kernel,
        grid=(1,),
        in_specs=[pl.BlockSpec((AR, AC), lambda i: (0, 0))],
        out_specs=[
            pl.BlockSpec((AR, AC), lambda i: (0, 0)),
            pl.BlockSpec((1, NB), lambda i: (0, 0)),
            pl.BlockSpec((1, NB), lambda i: (0, 0)),
        ],
        out_shape=[
            jax.ShapeDtypeStruct((AR, AC), jnp.int32),
            jax.ShapeDtypeStruct((1, NB), jnp.int32),
            jax.ShapeDtypeStruct((1, NB), jnp.int32),
        ],
    )(ej)
    pos_flat = pos.reshape(NA)
    p0 = pos_flat[0::2].reshape(1, S)
    p1 = pos_flat[1::2].reshape(1, S)

    xg = _sc_scatter(x2, p0, p1)

    yg = pl.pallas_call(
        _gmm_kernel,
        grid_spec=pltpu.PrefetchScalarGridSpec(
            num_scalar_prefetch=2,
            grid=(NB,),
            in_specs=[
                pl.BlockSpec((BME, D2), lambda i, be_, act_: (i, 0)),
                pl.BlockSpec((1, D, FF),
                             lambda i, be_, act_: (be_[i], 0, 0)),
                pl.BlockSpec((1, D, FF),
                             lambda i, be_, act_: (be_[i], 0, 0)),
                pl.BlockSpec((1, FF, D),
                             lambda i, be_, act_: (be_[i], 0, 0)),
            ],
            out_specs=pl.BlockSpec((BME, D2), lambda i, be_, act_: (i, 0)),
        ),
        out_shape=jax.ShapeDtypeStruct((P, D2), jnp.int32),
        compiler_params=pltpu.CompilerParams(
            dimension_semantics=("arbitrary",)),
    )(be.reshape(NB), act.reshape(NB), xg, wg, wu, wd)

    a_rows, b_rows = _sc_gather(yg, p0, p1)

    out = pl.pallas_call(
        _final_kernel,
        grid=(NT,),
        in_specs=[
            pl.BlockSpec((BT, D), lambda i: (i, 0)),
            pl.BlockSpec((BT, D2), lambda i: (i, 0)),
            pl.BlockSpec((BT, D2), lambda i: (i, 0)),
            pl.BlockSpec((BT, TOPK), lambda i: (i, 0)),
        ],
        out_specs=pl.BlockSpec((BT, D), lambda i: (i, 0)),
        out_shape=jax.ShapeDtypeStruct((s, d), jnp.float32),
        compiler_params=pltpu.CompilerParams(
            dimension_semantics=("parallel",)),
    )(h2, a_rows, b_rows, tw)

    return out.reshape(b, s, d)


# speedup vs baseline: 2.0584x; 1.0156x over previous
import jax
import jax.numpy as jnp
from jax.experimental import pallas as pl
from jax.experimental.pallas import tpu as pltpu
from jax.experimental.pallas import tpu_sc as plsc

S, D, H, E, FF, TOPK = 2048, 768, 12, 8, 1024, 2
DH = D // H
BT = 256
NT = S // BT
NEG = -1e30

NA = TOPK * S
AR, AC = 32, 128
BME = 256
P = 6144
NB = P // BME
WSC = 128


def _qkv_kernel(x_ref, ln1_ref, wq_ref, wk_ref, wv_ref, cos_ref, sin_ref,
                q_ref, k_ref, v_ref):
    x = x_ref[...]
    var = jnp.mean(jnp.square(x), axis=-1, keepdims=True)
    h = (x * jax.lax.rsqrt(var + 1e-6) * ln1_ref[...]).astype(jnp.bfloat16)
    q = jnp.dot(h, wq_ref[...], preferred_element_type=jnp.float32)
    k = jnp.dot(h, wk_ref[...], preferred_element_type=jnp.float32)
    v = jnp.dot(h, wv_ref[...], preferred_element_type=jnp.float32)
    cos = cos_ref[...]
    sin = sin_ref[...]
    col = jax.lax.broadcasted_iota(jnp.int32, (BT, D), 1)
    first_half = (col % DH) < (DH // 2)

    def rope(u):
        rot = jnp.where(first_half,
                        -pltpu.roll(u, D - DH // 2, 1),
                        pltpu.roll(u, DH // 2, 1))
        return u * cos + rot * sin

    q_ref[...] = (rope(q) * (1.0 / (DH ** 0.5))).astype(jnp.bfloat16)
    k_ref[...] = rope(k).astype(jnp.bfloat16)
    v_ref[...] = v.astype(jnp.bfloat16)


BQ = 512
NQ = S // BQ


def _attn_kernel(q_ref, k_ref, v_ref, o_ref):
    qi = pl.program_id(1)
    qs = (q_ref[:, :DH], q_ref[:, DH:])

    def step(kb, carry, masked):
        kk = k_ref[pl.ds(kb * BQ, BQ), :]
        vv = v_ref[pl.ds(kb * BQ, BQ), :]
        if masked:
            rows = jax.lax.broadcasted_iota(jnp.int32, (BQ, BQ), 0)
            cols = jax.lax.broadcasted_iota(jnp.int32, (BQ, BQ), 1)
            vis = rows >= cols
        out = []
        for hh in range(2):
            m, l, acc = carry[hh]
            k = kk[:, hh * DH:(hh + 1) * DH]
            s = jax.lax.dot_general(qs[hh], k, (((1,), (1,)), ((), ())),
                                    preferred_element_type=jnp.float32)
            if masked:
                s = jnp.where(vis, s, NEG)
            m_new = jnp.maximum(m, jnp.max(s, axis=1, keepdims=True))
            alpha = jnp.exp(m - m_new)
            p = jnp.exp(s - m_new)
            l = l * alpha + jnp.sum(p, axis=1, keepdims=True)
            v = vv[:, hh * DH:(hh + 1) * DH]
            pv = jnp.dot(p.astype(jnp.bfloat16), v,
                         preferred_element_type=jnp.float32)
            acc = acc * alpha + pv
            out.append((m_new, l, acc))
        return tuple(out)

    init = tuple((jnp.full((BQ, 1), NEG, jnp.float32),
                  jnp.zeros((BQ, 1), jnp.float32),
                  jnp.zeros((BQ, DH), jnp.float32)) for _ in range(2))
    carry = jax.lax.fori_loop(0, qi, lambda kb, c: step(kb, c, False), init)
    res = step(qi, carry, True)
    o_ref[...] = jnp.concatenate(
        [(acc / l).astype(jnp.bfloat16) for (m, l, acc) in res], axis=1)


def _post_attn_kernel(ctx_ref, wo_ref, dec_ref, ln2_ref, wr_ref,
                      h2_ref, x2_ref, ti_ref, tw_ref):
    ctx = ctx_ref[...]
    h2 = dec_ref[...] + jnp.dot(ctx, wo_ref[...],
                                preferred_element_type=jnp.float32)
    h2_ref[...] = h2
    var = jnp.mean(jnp.square(h2), axis=-1, keepdims=True)
    x2 = h2 * jax.lax.rsqrt(var + 1e-6) * ln2_ref[...]
    x2_ref[...] = _pack(x2.astype(jnp.bfloat16))
    logits = jax.lax.dot_general(x2, wr_ref[...], (((1,), (0,)), ((), ())),
                                 precision=jax.lax.Precision.HIGHEST,
                                 preferred_element_type=jnp.float32)
    col = jax.lax.broadcasted_iota(jnp.int32, (BT, E), 1)
    m1 = jnp.max(logits, axis=1, keepdims=True)
    i1 = jnp.min(jnp.where(logits == m1, col, E), axis=1, keepdims=True)
    masked = jnp.where(col == i1, NEG, logits)
    m2 = jnp.max(masked, axis=1, keepdims=True)
    i2 = jnp.min(jnp.where(masked == m2, col, E), axis=1, keepdims=True)
    w1 = 1.0 / (1.0 + jnp.exp(m2 - m1))
    w2 = 1.0 - w1
    two = jax.lax.broadcasted_iota(jnp.int32, (BT, TOPK), 1)
    ti_ref[...] = jnp.where(two == 0, i1, i2)
    tw_ref[...] = jnp.where(two == 0, w1, w2)


D2 = D // 2


def _pack(xb):
    lo = jax.lax.bitcast_convert_type(xb[:, :D2], jnp.uint16)
    hi = jax.lax.bitcast_convert_type(xb[:, D2:], jnp.uint16)
    u = (hi.astype(jnp.uint32) << 16) | lo.astype(jnp.uint32)
    return jax.lax.bitcast_convert_type(u, jnp.int32)


def _unpack(p):
    u = jax.lax.bitcast_convert_type(p, jnp.uint32)
    lo = jax.lax.bitcast_convert_type((u & 0xffff).astype(jnp.uint16),
                                      jnp.bfloat16)
    hi = jax.lax.bitcast_convert_type((u >> 16).astype(jnp.uint16),
                                      jnp.bfloat16)
    return jnp.concatenate([lo, hi], axis=1)


def _dispatch_kernel(ej_ref, pos_ref, be_ref, act_ref):
    ej = ej_ref[...]
    lane = jax.lax.broadcasted_iota(jnp.int32, (AR, AC), 1)
    srow = jax.lax.broadcasted_iota(jnp.int32, (AR, 1), 0)
    nb_iota = jax.lax.broadcasted_iota(jnp.int32, (1, NB), 1)
    pos = jnp.zeros((AR, AC), jnp.int32)
    be = jnp.zeros((1, NB), jnp.int32)
    off = jnp.int32(0)
    for e in range(E):
        m = (ej == e).astype(jnp.int32)
        pr = m
        for sh in (1, 2, 4, 8, 16, 32, 64):
            pr = pr + jnp.where(lane >= sh, pltpu.roll(pr, sh, 1), 0)
        rowtot = jnp.sum(m, axis=1, keepdims=True)
        rp = rowtot
        for sh in (1, 2, 4, 8, 16):
            rp = rp + jnp.where(srow >= sh, pltpu.roll(rp, sh, 0), 0)
        rank = (pr - m) + (rp - rowtot)
        cnt = jnp.sum(m)
        padded = ((cnt + BME - 1) // BME) * BME
        pos = jnp.where(ej == e, off + rank, pos)
        start_b = off // BME
        nblk = padded // BME
        be = jnp.where((nb_iota >= start_b) & (nb_iota < start_b + nblk),
                       e, be)
        off = off + padded
    pos_ref[...] = pos
    be_ref[...] = be
    act_ref[...] = (nb_iota < off // BME).astype(jnp.int32)


def _gmm_kernel(be_ref, act_ref, xg_ref, wg_ref, wu_ref, wd_ref, yg_ref):
    @pl.when(act_ref[pl.program_id(0)] != 0)
    def _():
        x = _unpack(xg_ref[...])
        g = jnp.dot(x, wg_ref[0], preferred_element_type=jnp.float32)
        u = jnp.dot(x, wu_ref[0], preferred_element_type=jnp.float32)
        act = (g * jax.nn.sigmoid(g) * u).astype(jnp.bfloat16)
        eo = jnp.dot(act, wd_ref[0], preferred_element_type=jnp.float32)
        yg_ref[...] = _pack(eo.astype(jnp.bfloat16))


def _final_kernel(h2_ref, a_ref, b_ref, tw_ref, out_ref):
    tw = tw_ref[...]
    w0 = tw[:, 0:1]
    w1 = tw[:, 1:2]
    out_ref[...] = (h2_ref[...]
                    + w0 * _unpack(a_ref[...]).astype(jnp.float32)
                    + w1 * _unpack(b_ref[...]).astype(jnp.float32))


def _sc_mesh():
    return plsc.VectorSubcoreMesh(core_axis_name="c", subcore_axis_name="s")


def _sc_scatter(x2i, p0, p1):
    @pl.kernel(out_type=jax.ShapeDtypeStruct((P, D2), jnp.int32),
               mesh=_sc_mesh())
    def scat(x2_hbm, p0_hbm, p1_hbm, xg_hbm):
        def body(x_vmem, i0_vmem, i1_vmem):
            pltpu.sync_copy(x_vmem, xg_hbm.at[i0_vmem.at[0]])
            pltpu.sync_copy(x_vmem, xg_hbm.at[i1_vmem.at[0]])

        pltpu.emit_pipeline(
            body,
            grid=(S // WSC,),
            in_specs=[pl.BlockSpec((WSC, D2), lambda i: (i, 0)),
                      pl.BlockSpec((1, WSC), lambda i: (0, i)),
                      pl.BlockSpec((1, WSC), lambda i: (0, i))],
            out_specs=[],
            core_axis_name=("c", "s"),
            dimension_semantics=(pltpu.PARALLEL,),
        )(x2_hbm, p0_hbm, p1_hbm)

    return scat(x2i, p0, p1)


def _sc_gather(ygi, p01):
    @pl.kernel(out_type=jax.ShapeDtypeStruct((2 * S, D2), jnp.int32),
               mesh=_sc_mesh())
    def gath(yg_hbm, p01_hbm, ab_hbm):
        def body(idx_vmem, ab_vmem):
            pltpu.sync_copy(yg_hbm.at[idx_vmem.at[0]], ab_vmem)

        pltpu.emit_pipeline(
            body,
            grid=(2 * S // WSC,),
            in_specs=[pl.BlockSpec((1, WSC), lambda i: (0, i))],
            out_specs=[pl.BlockSpec((WSC, D2), lambda i: (i, 0))],
            core_axis_name=("c", "s"),
            dimension_semantics=(pltpu.PARALLEL,),
        )(p01_hbm, ab_hbm)

    return gath(ygi, p01)


def kernel(dec_inp, ln1_w, ln2_w, Wq, Wk, Wv, Wo, Wrouter, Wgate, Wup, Wdown):
    b, s, d = dec_inp.shape
    x = dec_inp.reshape(s, d)
    ln1 = ln1_w.reshape(1, d)
    ln2 = ln2_w.reshape(1, d)
    wq = Wq.astype(jnp.bfloat16)
    wk = Wk.astype(jnp.bfloat16)
    wv = Wv.astype(jnp.bfloat16)
    wo = Wo.astype(jnp.bfloat16)
    wg = Wgate.astype(jnp.bfloat16)
    wu = Wup.astype(jnp.bfloat16)
    wd = Wdown.astype(jnp.bfloat16)

    inv_freq = 1.0 / (10000.0 ** (jnp.arange(0, DH, 2, dtype=jnp.float32) / DH))
    t = jnp.arange(s, dtype=jnp.float32)
    freqs = jnp.outer(t, inv_freq)
    emb = jnp.concatenate([freqs, freqs], axis=-1)
    cos = jnp.tile(jnp.cos(emb), (1, H))
    sin = jnp.tile(jnp.sin(emb), (1, H))

    bf = jnp.bfloat16
    q, k, v = pl.pallas_call(
        _qkv_kernel,
        grid=(NT,),
        in_specs=[
            pl.BlockSpec((BT, D), lambda i: (i, 0)),
            pl.BlockSpec((1, D), lambda i: (0, 0)),
            pl.BlockSpec((D, D), lambda i: (0, 0)),
            pl.BlockSpec((D, D), lambda i: (0, 0)),
            pl.BlockSpec((D, D), lambda i: (0, 0)),
            pl.BlockSpec((BT, D), lambda i: (i, 0)),
            pl.BlockSpec((BT, D), lambda i: (i, 0)),
        ],
        out_specs=[
            pl.BlockSpec((BT, D), lambda i: (i, 0)),
            pl.BlockSpec((BT, D), lambda i: (i, 0)),
            pl.BlockSpec((BT, D), lambda i: (i, 0)),
        ],
        out_shape=[jax.ShapeDtypeStruct((s, d), bf)] * 3,
        compiler_params=pltpu.CompilerParams(
            dimension_semantics=("parallel",)),
    )(x, ln1, wq, wk, wv, cos, sin)

    ctx = pl.pallas_call(
        _attn_kernel,
        grid=(H // 2, NQ),
        in_specs=[
            pl.BlockSpec((BQ, 2 * DH), lambda h, i: (i, h)),
            pl.BlockSpec((S, 2 * DH), lambda h, i: (0, h)),
            pl.BlockSpec((S, 2 * DH), lambda h, i: (0, h)),
        ],
        out_specs=pl.BlockSpec((BQ, 2 * DH), lambda h, i: (i, h)),
        out_shape=jax.ShapeDtypeStruct((s, d), bf),
        compiler_params=pltpu.CompilerParams(
            dimension_semantics=("parallel", "arbitrary")),
    )(q, k, v)

    h2, x2, ti, tw = pl.pallas_call(
        _post_attn_kernel,
        grid=(NT,),
        in_specs=[
            pl.BlockSpec((BT, D), lambda i: (i, 0)),
            pl.BlockSpec((D, D), lambda i: (0, 0)),
            pl.BlockSpec((BT, D), lambda i: (i, 0)),
            pl.BlockSpec((1, D), lambda i: (0, 0)),
            pl.BlockSpec((D, E), lambda i: (0, 0)),
        ],
        out_specs=[
            pl.BlockSpec((BT, D), lambda i: (i, 0)),
            pl.BlockSpec((BT, D2), lambda i: (i, 0)),
            pl.BlockSpec((BT, TOPK), lambda i: (i, 0)),
            pl.BlockSpec((BT, TOPK), lambda i: (i, 0)),
        ],
        out_shape=[
            jax.ShapeDtypeStruct((s, d), jnp.float32),
            jax.ShapeDtypeStruct((s, D2), jnp.int32),
            jax.ShapeDtypeStruct((s, TOPK), jnp.int32),
            jax.ShapeDtypeStruct((s, TOPK), jnp.float32),
        ],
        compiler_params=pltpu.CompilerParams(
            dimension_semantics=("parallel",)),
    )(ctx, wo, x, ln2, Wrouter)

    ej = ti.reshape(AR, AC)
    pos, be, act = pl.pallas_call(
        _dispatch_kernel,
        grid=(1,),
        in_specs=[pl.BlockSpec((AR, AC), lambda i: (0, 0))],
        out_specs=[
            pl.BlockSpec((AR, AC), lambda i: (0, 0)),
            pl.BlockSpec((1, NB), lambda i: (0, 0)),
            pl.BlockSpec((1, NB), lambda i: (0, 0)),
        ],
        out_shape=[
            jax.ShapeDtypeStruct((AR, AC), jnp.int32),
            jax.ShapeDtypeStruct((1, NB), jnp.int32),
            jax.ShapeDtypeStruct((1, NB), jnp.int32),
        ],
    )(ej)
    pos_flat = pos.reshape(NA)
    p0 = pos_flat[0::2].reshape(1, S)
    p1 = pos_flat[1::2].reshape(1, S)

    xg = _sc_scatter(x2, p0, p1)

    yg = pl.pallas_call(
        _gmm_kernel,
        grid_spec=pltpu.PrefetchScalarGridSpec(
            num_scalar_prefetch=2,
            grid=(NB,),
            in_specs=[
                pl.BlockSpec((BME, D2), lambda i, be_, act_: (i, 0)),
                pl.BlockSpec((1, D, FF),
                             lambda i, be_, act_: (be_[i], 0, 0)),
                pl.BlockSpec((1, D, FF),
                             lambda i, be_, act_: (be_[i], 0, 0)),
                pl.BlockSpec((1, FF, D),
                             lambda i, be_, act_: (be_[i], 0, 0)),
            ],
            out_specs=pl.BlockSpec((BME, D2), lambda i, be_, act_: (i, 0)),
        ),
        out_shape=jax.ShapeDtypeStruct((P, D2), jnp.int32),
        compiler_params=pltpu.CompilerParams(
            dimension_semantics=("arbitrary",)),
    )(be.reshape(NB), act.reshape(NB), xg, wg, wu, wd)

    ab = _sc_gather(yg, jnp.concatenate([p0, p1], axis=1))

    out = pl.pallas_call(
        _final_kernel,
        grid=(NT,),
        in_specs=[
            pl.BlockSpec((BT, D), lambda i: (i, 0)),
            pl.BlockSpec((BT, D2), lambda i: (i, 0)),
            pl.BlockSpec((BT, D2), lambda i: (i + NT, 0)),
            pl.BlockSpec((BT, TOPK), lambda i: (i, 0)),
        ],
        out_specs=pl.BlockSpec((BT, D), lambda i: (i, 0)),
        out_shape=jax.ShapeDtypeStruct((s, d), jnp.float32),
        compiler_params=pltpu.CompilerParams(
            dimension_semantics=("parallel",)),
    )(h2, ab, ab, tw)

    return out.reshape(b, s, d)


# speedup vs baseline: 2.2060x; 1.0717x over previous
import jax
import jax.numpy as jnp
from jax.experimental import pallas as pl
from jax.experimental.pallas import tpu as pltpu
from jax.experimental.pallas import tpu_sc as plsc

S, D, H, E, FF, TOPK = 2048, 768, 12, 8, 1024, 2
DH = D // H
BT = 256
NT = S // BT
NEG = -1e30

NA = TOPK * S
AR, AC = 32, 128
BME = 256
P = 6144
NB = P // BME
WSC = 128


def _qkv_kernel(x_ref, ln1_ref, wq_ref, wk_ref, wv_ref, cos_ref, sin_ref,
                q_ref, k_ref, v_ref):
    x = x_ref[...]
    var = jnp.mean(jnp.square(x), axis=-1, keepdims=True)
    h = (x * jax.lax.rsqrt(var + 1e-6) * ln1_ref[...]).astype(jnp.bfloat16)
    q = jnp.dot(h, wq_ref[...], preferred_element_type=jnp.float32)
    k = jnp.dot(h, wk_ref[...], preferred_element_type=jnp.float32)
    v = jnp.dot(h, wv_ref[...], preferred_element_type=jnp.float32)
    cos = cos_ref[...]
    sin = sin_ref[...]
    col = jax.lax.broadcasted_iota(jnp.int32, (BT, D), 1)
    first_half = (col % DH) < (DH // 2)

    def rope(u):
        rot = jnp.where(first_half,
                        -pltpu.roll(u, D - DH // 2, 1),
                        pltpu.roll(u, DH // 2, 1))
        return u * cos + rot * sin

    q_ref[...] = (rope(q) * (1.0 / (DH ** 0.5))).astype(jnp.bfloat16)
    k_ref[...] = rope(k).astype(jnp.bfloat16)
    v_ref[...] = v.astype(jnp.bfloat16)


BQ = 1024
NQ = S // BQ


def _attn_kernel(q_ref, k_ref, v_ref, o_ref):
    qi = pl.program_id(1)
    qs = (q_ref[:, :DH], q_ref[:, DH:])

    def step(kb, carry, masked):
        kk = k_ref[pl.ds(kb * BQ, BQ), :]
        vv = v_ref[pl.ds(kb * BQ, BQ), :]
        if masked:
            rows = jax.lax.broadcasted_iota(jnp.int32, (BQ, BQ), 0)
            cols = jax.lax.broadcasted_iota(jnp.int32, (BQ, BQ), 1)
            vis = rows >= cols
        out = []
        for hh in range(2):
            m, l, acc = carry[hh]
            k = kk[:, hh * DH:(hh + 1) * DH]
            s = jax.lax.dot_general(qs[hh], k, (((1,), (1,)), ((), ())),
                                    preferred_element_type=jnp.float32)
            if masked:
                s = jnp.where(vis, s, NEG)
            m_new = jnp.maximum(m, jnp.max(s, axis=1, keepdims=True))
            alpha = jnp.exp(m - m_new)
            p = jnp.exp(s - m_new)
            l = l * alpha + jnp.sum(p, axis=1, keepdims=True)
            v = vv[:, hh * DH:(hh + 1) * DH]
            pv = jnp.dot(p.astype(jnp.bfloat16), v,
                         preferred_element_type=jnp.float32)
            acc = acc * alpha + pv
            out.append((m_new, l, acc))
        return tuple(out)

    init = tuple((jnp.full((BQ, 1), NEG, jnp.float32),
                  jnp.zeros((BQ, 1), jnp.float32),
                  jnp.zeros((BQ, DH), jnp.float32)) for _ in range(2))
    carry = jax.lax.fori_loop(0, qi, lambda kb, c: step(kb, c, False), init)
    res = step(qi, carry, True)
    o_ref[...] = jnp.concatenate(
        [(acc / l).astype(jnp.bfloat16) for (m, l, acc) in res], axis=1)


def _post_attn_kernel(ctx_ref, wo_ref, dec_ref, ln2_ref, wr_ref,
                      h2_ref, x2_ref, ti_ref, tw_ref):
    ctx = ctx_ref[...]
    h2 = dec_ref[...] + jnp.dot(ctx, wo_ref[...],
                                preferred_element_type=jnp.float32)
    h2_ref[...] = h2
    var = jnp.mean(jnp.square(h2), axis=-1, keepdims=True)
    x2 = h2 * jax.lax.rsqrt(var + 1e-6) * ln2_ref[...]
    x2_ref[...] = _pack(x2.astype(jnp.bfloat16))
    logits = jax.lax.dot_general(x2, wr_ref[...], (((1,), (0,)), ((), ())),
                                 precision=jax.lax.Precision.HIGHEST,
                                 preferred_element_type=jnp.float32)
    col = jax.lax.broadcasted_iota(jnp.int32, (BT, E), 1)
    m1 = jnp.max(logits, axis=1, keepdims=True)
    i1 = jnp.min(jnp.where(logits == m1, col, E), axis=1, keepdims=True)
    masked = jnp.where(col == i1, NEG, logits)
    m2 = jnp.max(masked, axis=1, keepdims=True)
    i2 = jnp.min(jnp.where(masked == m2, col, E), axis=1, keepdims=True)
    w1 = 1.0 / (1.0 + jnp.exp(m2 - m1))
    w2 = 1.0 - w1
    two = jax.lax.broadcasted_iota(jnp.int32, (BT, TOPK), 1)
    ti_ref[...] = jnp.where(two == 0, i1, i2)
    tw_ref[...] = jnp.where(two == 0, w1, w2)


D2 = D // 2


def _pack(xb):
    lo = jax.lax.bitcast_convert_type(xb[:, :D2], jnp.uint16)
    hi = jax.lax.bitcast_convert_type(xb[:, D2:], jnp.uint16)
    u = (hi.astype(jnp.uint32) << 16) | lo.astype(jnp.uint32)
    return jax.lax.bitcast_convert_type(u, jnp.int32)


def _unpack(p):
    u = jax.lax.bitcast_convert_type(p, jnp.uint32)
    lo = jax.lax.bitcast_convert_type((u & 0xffff).astype(jnp.uint16),
                                      jnp.bfloat16)
    hi = jax.lax.bitcast_convert_type((u >> 16).astype(jnp.uint16),
                                      jnp.bfloat16)
    return jnp.concatenate([lo, hi], axis=1)


def _dispatch_kernel(ej_ref, pos_ref, be_ref, act_ref):
    ej = ej_ref[...]
    lane = jax.lax.broadcasted_iota(jnp.int32, (AR, AC), 1)
    srow = jax.lax.broadcasted_iota(jnp.int32, (AR, 1), 0)
    nb_iota = jax.lax.broadcasted_iota(jnp.int32, (1, NB), 1)
    pos = jnp.zeros((AR, AC), jnp.int32)
    be = jnp.zeros((1, NB), jnp.int32)
    off = jnp.int32(0)
    for e in range(E):
        m = (ej == e).astype(jnp.int32)
        pr = m
        for sh in (1, 2, 4, 8, 16, 32, 64):
            pr = pr + jnp.where(lane >= sh, pltpu.roll(pr, sh, 1), 0)
        rowtot = jnp.sum(m, axis=1, keepdims=True)
        rp = rowtot
        for sh in (1, 2, 4, 8, 16):
            rp = rp + jnp.where(srow >= sh, pltpu.roll(rp, sh, 0), 0)
        rank = (pr - m) + (rp - rowtot)
        cnt = jnp.sum(m)
        padded = ((cnt + BME - 1) // BME) * BME
        pos = jnp.where(ej == e, off + rank, pos)
        start_b = off // BME
        nblk = padded // BME
        be = jnp.where((nb_iota >= start_b) & (nb_iota < start_b + nblk),
                       e, be)
        off = off + padded
    pos_ref[...] = pos
    be_ref[...] = be
    act_ref[...] = (nb_iota < off // BME).astype(jnp.int32)


def _gmm_kernel(be_ref, act_ref, xg_ref, wg_ref, wu_ref, wd_ref, yg_ref):
    @pl.when(act_ref[pl.program_id(0)] != 0)
    def _():
        x = _unpack(xg_ref[...])
        g = jnp.dot(x, wg_ref[0], preferred_element_type=jnp.float32)
        u = jnp.dot(x, wu_ref[0], preferred_element_type=jnp.float32)
        act = (g * jax.nn.sigmoid(g) * u).astype(jnp.bfloat16)
        eo = jnp.dot(act, wd_ref[0], preferred_element_type=jnp.float32)
        yg_ref[...] = _pack(eo.astype(jnp.bfloat16))


def _final_kernel(h2_ref, a_ref, b_ref, tw_ref, out_ref):
    tw = tw_ref[...]
    w0 = tw[:, 0:1]
    w1 = tw[:, 1:2]
    out_ref[...] = (h2_ref[...]
                    + w0 * _unpack(a_ref[...]).astype(jnp.float32)
                    + w1 * _unpack(b_ref[...]).astype(jnp.float32))


def _sc_mesh():
    return plsc.VectorSubcoreMesh(core_axis_name="c", subcore_axis_name="s")


def _sc_scatter(x2i, p0, p1):
    @pl.kernel(out_type=jax.ShapeDtypeStruct((P, D2), jnp.int32),
               mesh=_sc_mesh())
    def scat(x2_hbm, p0_hbm, p1_hbm, xg_hbm):
        def body(x_vmem, i0_vmem, i1_vmem):
            pltpu.sync_copy(x_vmem, xg_hbm.at[i0_vmem.at[0]])
            pltpu.sync_copy(x_vmem, xg_hbm.at[i1_vmem.at[0]])

        pltpu.emit_pipeline(
            body,
            grid=(S // WSC,),
            in_specs=[pl.BlockSpec((WSC, D2), lambda i: (i, 0)),
                      pl.BlockSpec((1, WSC), lambda i: (0, i)),
                      pl.BlockSpec((1, WSC), lambda i: (0, i))],
            out_specs=[],
            core_axis_name=("c", "s"),
            dimension_semantics=(pltpu.PARALLEL,),
        )(x2_hbm, p0_hbm, p1_hbm)

    return scat(x2i, p0, p1)


def _sc_gather(ygi, p01):
    @pl.kernel(out_type=jax.ShapeDtypeStruct((2 * S, D2), jnp.int32),
               mesh=_sc_mesh())
    def gath(yg_hbm, p01_hbm, ab_hbm):
        def body(idx_vmem, ab_vmem):
            pltpu.sync_copy(yg_hbm.at[idx_vmem.at[0]], ab_vmem)

        pltpu.emit_pipeline(
            body,
            grid=(2 * S // WSC,),
            in_specs=[pl.BlockSpec((1, WSC), lambda i: (0, i))],
            out_specs=[pl.BlockSpec((WSC, D2), lambda i: (i, 0))],
            core_axis_name=("c", "s"),
            dimension_semantics=(pltpu.PARALLEL,),
        )(p01_hbm, ab_hbm)

    return gath(ygi, p01)


def kernel(dec_inp, ln1_w, ln2_w, Wq, Wk, Wv, Wo, Wrouter, Wgate, Wup, Wdown):
    b, s, d = dec_inp.shape
    x = dec_inp.reshape(s, d)
    ln1 = ln1_w.reshape(1, d)
    ln2 = ln2_w.reshape(1, d)
    wq = Wq.astype(jnp.bfloat16)
    wk = Wk.astype(jnp.bfloat16)
    wv = Wv.astype(jnp.bfloat16)
    wo = Wo.astype(jnp.bfloat16)
    wg = Wgate.astype(jnp.bfloat16)
    wu = Wup.astype(jnp.bfloat16)
    wd = Wdown.astype(jnp.bfloat16)

    inv_freq = 1.0 / (10000.0 ** (jnp.arange(0, DH, 2, dtype=jnp.float32) / DH))
    t = jnp.arange(s, dtype=jnp.float32)
    freqs = jnp.outer(t, inv_freq)
    emb = jnp.concatenate([freqs, freqs], axis=-1)
    cos = jnp.tile(jnp.cos(emb), (1, H))
    sin = jnp.tile(jnp.sin(emb), (1, H))

    bf = jnp.bfloat16
    q, k, v = pl.pallas_call(
        _qkv_kernel,
        grid=(NT,),
        in_specs=[
            pl.BlockSpec((BT, D), lambda i: (i, 0)),
            pl.BlockSpec((1, D), lambda i: (0, 0)),
            pl.BlockSpec((D, D), lambda i: (0, 0)),
            pl.BlockSpec((D, D), lambda i: (0, 0)),
            pl.BlockSpec((D, D), lambda i: (0, 0)),
            pl.BlockSpec((BT, D), lambda i: (i, 0)),
            pl.BlockSpec((BT, D), lambda i: (i, 0)),
        ],
        out_specs=[
            pl.BlockSpec((BT, D), lambda i: (i, 0)),
            pl.BlockSpec((BT, D), lambda i: (i, 0)),
            pl.BlockSpec((BT, D), lambda i: (i, 0)),
        ],
        out_shape=[jax.ShapeDtypeStruct((s, d), bf)] * 3,
        compiler_params=pltpu.CompilerParams(
            dimension_semantics=("parallel",)),
    )(x, ln1, wq, wk, wv, cos, sin)

    ctx = pl.pallas_call(
        _attn_kernel,
        grid=(H // 2, NQ),
        in_specs=[
            pl.BlockSpec((BQ, 2 * DH), lambda h, i: (i, h)),
            pl.BlockSpec((S, 2 * DH), lambda h, i: (0, h)),
            pl.BlockSpec((S, 2 * DH), lambda h, i: (0, h)),
        ],
        out_specs=pl.BlockSpec((BQ, 2 * DH), lambda h, i: (i, h)),
        out_shape=jax.ShapeDtypeStruct((s, d), bf),
        compiler_params=pltpu.CompilerParams(
            dimension_semantics=("parallel", "arbitrary")),
    )(q, k, v)

    h2, x2, ti, tw = pl.pallas_call(
        _post_attn_kernel,
        grid=(NT,),
        in_specs=[
            pl.BlockSpec((BT, D), lambda i: (i, 0)),
            pl.BlockSpec((D, D), lambda i: (0, 0)),
            pl.BlockSpec((BT, D), lambda i: (i, 0)),
            pl.BlockSpec((1, D), lambda i: (0, 0)),
            pl.BlockSpec((D, E), lambda i: (0, 0)),
        ],
        out_specs=[
            pl.BlockSpec((BT, D), lambda i: (i, 0)),
            pl.BlockSpec((BT, D2), lambda i: (i, 0)),
            pl.BlockSpec((BT, TOPK), lambda i: (i, 0)),
            pl.BlockSpec((BT, TOPK), lambda i: (i, 0)),
        ],
        out_shape=[
            jax.ShapeDtypeStruct((s, d), jnp.float32),
            jax.ShapeDtypeStruct((s, D2), jnp.int32),
            jax.ShapeDtypeStruct((s, TOPK), jnp.int32),
            jax.ShapeDtypeStruct((s, TOPK), jnp.float32),
        ],
        compiler_params=pltpu.CompilerParams(
            dimension_semantics=("parallel",)),
    )(ctx, wo, x, ln2, Wrouter)

    ej = ti.reshape(AR, AC)
    pos, be, act = pl.pallas_call(
        _dispatch_kernel,
        grid=(1,),
        in_specs=[pl.BlockSpec((AR, AC), lambda i: (0, 0))],
        out_specs=[
            pl.BlockSpec((AR, AC), lambda i: (0, 0)),
            pl.BlockSpec((1, NB), lambda i: (0, 0)),
            pl.BlockSpec((1, NB), lambda i: (0, 0)),
        ],
        out_shape=[
            jax.ShapeDtypeStruct((AR, AC), jnp.int32),
            jax.ShapeDtypeStruct((1, NB), jnp.int32),
            jax.ShapeDtypeStruct((1, NB), jnp.int32),
        ],
    )(ej)
    pos_flat = pos.reshape(NA)
    p0 = pos_flat[0::2].reshape(1, S)
    p1 = pos_flat[1::2].reshape(1, S)

    xg = _sc_scatter(x2, p0, p1)

    yg = pl.pallas_call(
        _gmm_kernel,
        grid_spec=pltpu.PrefetchScalarGridSpec(
            num_scalar_prefetch=2,
            grid=(NB,),
            in_specs=[
                pl.BlockSpec((BME, D2), lambda i, be_, act_: (i, 0)),
                pl.BlockSpec((1, D, FF),
                             lambda i, be_, act_: (be_[i], 0, 0)),
                pl.BlockSpec((1, D, FF),
                             lambda i, be_, act_: (be_[i], 0, 0)),
                pl.BlockSpec((1, FF, D),
                             lambda i, be_, act_: (be_[i], 0, 0)),
            ],
            out_specs=pl.BlockSpec((BME, D2), lambda i, be_, act_: (i, 0)),
        ),
        out_shape=jax.ShapeDtypeStruct((P, D2), jnp.int32),
        compiler_params=pltpu.CompilerParams(
            dimension_semantics=("arbitrary",)),
    )(be.reshape(NB), act.reshape(NB), xg, wg, wu, wd)

    ab = _sc_gather(yg, jnp.concatenate([p0, p1], axis=1))

    out = pl.pallas_call(
        _final_kernel,
        grid=(NT,),
        in_specs=[
            pl.BlockSpec((BT, D), lambda i: (i, 0)),
            pl.BlockSpec((BT, D2), lambda i: (i, 0)),
            pl.BlockSpec((BT, D2), lambda i: (i + NT, 0)),
            pl.BlockSpec((BT, TOPK), lambda i: (i, 0)),
        ],
        out_specs=pl.BlockSpec((BT, D), lambda i: (i, 0)),
        out_shape=jax.ShapeDtypeStruct((s, d), jnp.float32),
        compiler_params=pltpu.CompilerParams(
            dimension_semantics=("parallel",)),
    )(h2, ab, ab, tw)

    return out.reshape(b, s, d)


# speedup vs baseline: 2.2552x; 1.0223x over previous
import jax
import jax.numpy as jnp
from jax.experimental import pallas as pl
from jax.experimental.pallas import tpu as pltpu
from jax.experimental.pallas import tpu_sc as plsc

S, D, H, E, FF, TOPK = 2048, 768, 12, 8, 1024, 2
DH = D // H
BT = 256
NT = S // BT
NEG = -1e30

NA = TOPK * S
AR, AC = 32, 128
BME = 512
P = 8192
NB = P // BME
WSC = 128


def _qkv_kernel(x_ref, ln1_ref, wq_ref, wk_ref, wv_ref, cos_ref, sin_ref,
                q_ref, k_ref, v_ref):
    x = x_ref[...]
    var = jnp.mean(jnp.square(x), axis=-1, keepdims=True)
    h = (x * jax.lax.rsqrt(var + 1e-6) * ln1_ref[...]).astype(jnp.bfloat16)
    q = jnp.dot(h, wq_ref[...], preferred_element_type=jnp.float32)
    k = jnp.dot(h, wk_ref[...], preferred_element_type=jnp.float32)
    v = jnp.dot(h, wv_ref[...], preferred_element_type=jnp.float32)
    cos = cos_ref[...]
    sin = sin_ref[...]
    col = jax.lax.broadcasted_iota(jnp.int32, (BT, D), 1)
    first_half = (col % DH) < (DH // 2)

    def rope(u):
        rot = jnp.where(first_half,
                        -pltpu.roll(u, D - DH // 2, 1),
                        pltpu.roll(u, DH // 2, 1))
        return u * cos + rot * sin

    q_ref[...] = (rope(q) * (1.0 / (DH ** 0.5))).astype(jnp.bfloat16)
    k_ref[...] = rope(k).astype(jnp.bfloat16)
    v_ref[...] = v.astype(jnp.bfloat16)


BQ = 1024
NQ = S // BQ


def _attn_kernel(q_ref, k_ref, v_ref, o_ref):
    qi = pl.program_id(1)
    qs = (q_ref[:, :DH], q_ref[:, DH:])

    def step(kb, carry, masked):
        kk = k_ref[pl.ds(kb * BQ, BQ), :]
        vv = v_ref[pl.ds(kb * BQ, BQ), :]
        if masked:
            rows = jax.lax.broadcasted_iota(jnp.int32, (BQ, BQ), 0)
            cols = jax.lax.broadcasted_iota(jnp.int32, (BQ, BQ), 1)
            vis = rows >= cols
        out = []
        for hh in range(2):
            m, l, acc = carry[hh]
            k = kk[:, hh * DH:(hh + 1) * DH]
            s = jax.lax.dot_general(qs[hh], k, (((1,), (1,)), ((), ())),
                                    preferred_element_type=jnp.float32)
            if masked:
                s = jnp.where(vis, s, NEG)
            m_new = jnp.maximum(m, jnp.max(s, axis=1, keepdims=True))
            alpha = jnp.exp(m - m_new)
            p = jnp.exp(s - m_new)
            l = l * alpha + jnp.sum(p, axis=1, keepdims=True)
            v = vv[:, hh * DH:(hh + 1) * DH]
            pv = jnp.dot(p.astype(jnp.bfloat16), v,
                         preferred_element_type=jnp.float32)
            acc = acc * alpha + pv
            out.append((m_new, l, acc))
        return tuple(out)

    init = tuple((jnp.full((BQ, 1), NEG, jnp.float32),
                  jnp.zeros((BQ, 1), jnp.float32),
                  jnp.zeros((BQ, DH), jnp.float32)) for _ in range(2))
    carry = jax.lax.fori_loop(0, qi, lambda kb, c: step(kb, c, False), init)
    res = step(qi, carry, True)
    o_ref[...] = jnp.concatenate(
        [(acc / l).astype(jnp.bfloat16) for (m, l, acc) in res], axis=1)


def _post_attn_kernel(ctx_ref, wo_ref, dec_ref, ln2_ref, wr_ref,
                      h2_ref, x2_ref, ti_ref, tw_ref):
    ctx = ctx_ref[...]
    h2 = dec_ref[...] + jnp.dot(ctx, wo_ref[...],
                                preferred_element_type=jnp.float32)
    h2_ref[...] = h2
    var = jnp.mean(jnp.square(h2), axis=-1, keepdims=True)
    x2 = h2 * jax.lax.rsqrt(var + 1e-6) * ln2_ref[...]
    x2_ref[...] = _pack(x2.astype(jnp.bfloat16))
    logits = jax.lax.dot_general(x2, wr_ref[...], (((1,), (0,)), ((), ())),
                                 precision=jax.lax.Precision.HIGHEST,
                                 preferred_element_type=jnp.float32)
    col = jax.lax.broadcasted_iota(jnp.int32, (BT, E), 1)
    m1 = jnp.max(logits, axis=1, keepdims=True)
    i1 = jnp.min(jnp.where(logits == m1, col, E), axis=1, keepdims=True)
    masked = jnp.where(col == i1, NEG, logits)
    m2 = jnp.max(masked, axis=1, keepdims=True)
    i2 = jnp.min(jnp.where(masked == m2, col, E), axis=1, keepdims=True)
    w1 = 1.0 / (1.0 + jnp.exp(m2 - m1))
    w2 = 1.0 - w1
    two = jax.lax.broadcasted_iota(jnp.int32, (BT, TOPK), 1)
    ti_ref[...] = jnp.where(two == 0, i1, i2)
    tw_ref[...] = jnp.where(two == 0, w1, w2)


D2 = D // 2


def _pack(xb):
    lo = jax.lax.bitcast_convert_type(xb[:, :D2], jnp.uint16)
    hi = jax.lax.bitcast_convert_type(xb[:, D2:], jnp.uint16)
    u = (hi.astype(jnp.uint32) << 16) | lo.astype(jnp.uint32)
    return jax.lax.bitcast_convert_type(u, jnp.int32)


def _unpack(p):
    u = jax.lax.bitcast_convert_type(p, jnp.uint32)
    lo = jax.lax.bitcast_convert_type((u & 0xffff).astype(jnp.uint16),
                                      jnp.bfloat16)
    hi = jax.lax.bitcast_convert_type((u >> 16).astype(jnp.uint16),
                                      jnp.bfloat16)
    return jnp.concatenate([lo, hi], axis=1)


def _dispatch_kernel(ej_ref, pos_ref, be_ref, act_ref):
    ej = ej_ref[...]
    lane = jax.lax.broadcasted_iota(jnp.int32, (AR, AC), 1)
    srow = jax.lax.broadcasted_iota(jnp.int32, (AR, 1), 0)
    nb_iota = jax.lax.broadcasted_iota(jnp.int32, (1, NB), 1)
    pos = jnp.zeros((AR, AC), jnp.int32)
    be = jnp.zeros((1, NB), jnp.int32)
    off = jnp.int32(0)
    for e in range(E):
        m = (ej == e).astype(jnp.int32)
        pr = m
        for sh in (1, 2, 4, 8, 16, 32, 64):
            pr = pr + jnp.where(lane >= sh, pltpu.roll(pr, sh, 1), 0)
        rowtot = jnp.sum(m, axis=1, keepdims=True)
        rp = rowtot
        for sh in (1, 2, 4, 8, 16):
            rp = rp + jnp.where(srow >= sh, pltpu.roll(rp, sh, 0), 0)
        rank = (pr - m) + (rp - rowtot)
        cnt = jnp.sum(m)
        padded = ((cnt + BME - 1) // BME) * BME
        pos = jnp.where(ej == e, off + rank, pos)
        start_b = off // BME
        nblk = padded // BME
        be = jnp.where((nb_iota >= start_b) & (nb_iota < start_b + nblk),
                       e, be)
        off = off + padded
    pos_ref[...] = pos
    be_ref[...] = be
    act_ref[...] = (nb_iota < off // BME).astype(jnp.int32)


def _gmm_kernel(be_ref, act_ref, xg_ref, wg_ref, wu_ref, wd_ref, yg_ref):
    @pl.when(act_ref[pl.program_id(0)] != 0)
    def _():
        x = _unpack(xg_ref[...])
        g = jnp.dot(x, wg_ref[0], preferred_element_type=jnp.float32)
        u = jnp.dot(x, wu_ref[0], preferred_element_type=jnp.float32)
        act = (g * jax.nn.sigmoid(g) * u).astype(jnp.bfloat16)
        eo = jnp.dot(act, wd_ref[0], preferred_element_type=jnp.float32)
        yg_ref[...] = _pack(eo.astype(jnp.bfloat16))


def _final_kernel(h2_ref, a_ref, b_ref, tw_ref, out_ref):
    tw = tw_ref[...]
    w0 = tw[:, 0:1]
    w1 = tw[:, 1:2]
    out_ref[...] = (h2_ref[...]
                    + w0 * _unpack(a_ref[...]).astype(jnp.float32)
                    + w1 * _unpack(b_ref[...]).astype(jnp.float32))


def _sc_mesh():
    return plsc.VectorSubcoreMesh(core_axis_name="c", subcore_axis_name="s")


def _sc_scatter(x2i, p0, p1):
    @pl.kernel(out_type=jax.ShapeDtypeStruct((P, D2), jnp.int32),
               mesh=_sc_mesh())
    def scat(x2_hbm, p0_hbm, p1_hbm, xg_hbm):
        def body(x_vmem, i0_vmem, i1_vmem):
            pltpu.sync_copy(x_vmem, xg_hbm.at[i0_vmem.at[0]])
            pltpu.sync_copy(x_vmem, xg_hbm.at[i1_vmem.at[0]])

        pltpu.emit_pipeline(
            body,
            grid=(S // WSC,),
            in_specs=[pl.BlockSpec((WSC, D2), lambda i: (i, 0)),
                      pl.BlockSpec((1, WSC), lambda i: (0, i)),
                      pl.BlockSpec((1, WSC), lambda i: (0, i))],
            out_specs=[],
            core_axis_name=("c", "s"),
            dimension_semantics=(pltpu.PARALLEL,),
        )(x2_hbm, p0_hbm, p1_hbm)

    return scat(x2i, p0, p1)


def _sc_gather(ygi, p01):
    @pl.kernel(out_type=jax.ShapeDtypeStruct((2 * S, D2), jnp.int32),
               mesh=_sc_mesh())
    def gath(yg_hbm, p01_hbm, ab_hbm):
        def body(idx_vmem, ab_vmem):
            pltpu.sync_copy(yg_hbm.at[idx_vmem.at[0]], ab_vmem)

        pltpu.emit_pipeline(
            body,
            grid=(2 * S // WSC,),
            in_specs=[pl.BlockSpec((1, WSC), lambda i: (0, i))],
            out_specs=[pl.BlockSpec((WSC, D2), lambda i: (i, 0))],
            core_axis_name=("c", "s"),
            dimension_semantics=(pltpu.PARALLEL,),
        )(p01_hbm, ab_hbm)

    return gath(ygi, p01)


def kernel(dec_inp, ln1_w, ln2_w, Wq, Wk, Wv, Wo, Wrouter, Wgate, Wup, Wdown):
    b, s, d = dec_inp.shape
    x = dec_inp.reshape(s, d)
    ln1 = ln1_w.reshape(1, d)
    ln2 = ln2_w.reshape(1, d)
    wq = Wq.astype(jnp.bfloat16)
    wk = Wk.astype(jnp.bfloat16)
    wv = Wv.astype(jnp.bfloat16)
    wo = Wo.astype(jnp.bfloat16)
    wg = Wgate.astype(jnp.bfloat16)
    wu = Wup.astype(jnp.bfloat16)
    wd = Wdown.astype(jnp.bfloat16)

    inv_freq = 1.0 / (10000.0 ** (jnp.arange(0, DH, 2, dtype=jnp.float32) / DH))
    t = jnp.arange(s, dtype=jnp.float32)
    freqs = jnp.outer(t, inv_freq)
    emb = jnp.concatenate([freqs, freqs], axis=-1)
    cos = jnp.tile(jnp.cos(emb), (1, H))
    sin = jnp.tile(jnp.sin(emb), (1, H))

    bf = jnp.bfloat16
    q, k, v = pl.pallas_call(
        _qkv_kernel,
        grid=(NT,),
        in_specs=[
            pl.BlockSpec((BT, D), lambda i: (i, 0)),
            pl.BlockSpec((1, D), lambda i: (0, 0)),
            pl.BlockSpec((D, D), lambda i: (0, 0)),
            pl.BlockSpec((D, D), lambda i: (0, 0)),
            pl.BlockSpec((D, D), lambda i: (0, 0)),
            pl.BlockSpec((BT, D), lambda i: (i, 0)),
            pl.BlockSpec((BT, D), lambda i: (i, 0)),
        ],
        out_specs=[
            pl.BlockSpec((BT, D), lambda i: (i, 0)),
            pl.BlockSpec((BT, D), lambda i: (i, 0)),
            pl.BlockSpec((BT, D), lambda i: (i, 0)),
        ],
        out_shape=[jax.ShapeDtypeStruct((s, d), bf)] * 3,
        compiler_params=pltpu.CompilerParams(
            dimension_semantics=("parallel",)),
    )(x, ln1, wq, wk, wv, cos, sin)

    ctx = pl.pallas_call(
        _attn_kernel,
        grid=(H // 2, NQ),
        in_specs=[
            pl.BlockSpec((BQ, 2 * DH), lambda h, i: (i, h)),
            pl.BlockSpec((S, 2 * DH), lambda h, i: (0, h)),
            pl.BlockSpec((S, 2 * DH), lambda h, i: (0, h)),
        ],
        out_specs=pl.BlockSpec((BQ, 2 * DH), lambda h, i: (i, h)),
        out_shape=jax.ShapeDtypeStruct((s, d), bf),
        compiler_params=pltpu.CompilerParams(
            dimension_semantics=("parallel", "arbitrary")),
    )(q, k, v)

    h2, x2, ti, tw = pl.pallas_call(
        _post_attn_kernel,
        grid=(NT,),
        in_specs=[
            pl.BlockSpec((BT, D), lambda i: (i, 0)),
            pl.BlockSpec((D, D), lambda i: (0, 0)),
            pl.BlockSpec((BT, D), lambda i: (i, 0)),
            pl.BlockSpec((1, D), lambda i: (0, 0)),
            pl.BlockSpec((D, E), lambda i: (0, 0)),
        ],
        out_specs=[
            pl.BlockSpec((BT, D), lambda i: (i, 0)),
            pl.BlockSpec((BT, D2), lambda i: (i, 0)),
            pl.BlockSpec((BT, TOPK), lambda i: (i, 0)),
            pl.BlockSpec((BT, TOPK), lambda i: (i, 0)),
        ],
        out_shape=[
            jax.ShapeDtypeStruct((s, d), jnp.float32),
            jax.ShapeDtypeStruct((s, D2), jnp.int32),
            jax.ShapeDtypeStruct((s, TOPK), jnp.int32),
            jax.ShapeDtypeStruct((s, TOPK), jnp.float32),
        ],
        compiler_params=pltpu.CompilerParams(
            dimension_semantics=("parallel",)),
    )(ctx, wo, x, ln2, Wrouter)

    ej = ti.reshape(AR, AC)
    pos, be, act = pl.pallas_call(
        _dispatch_kernel,
        grid=(1,),
        in_specs=[pl.BlockSpec((AR, AC), lambda i: (0, 0))],
        out_specs=[
            pl.BlockSpec((AR, AC), lambda i: (0, 0)),
            pl.BlockSpec((1, NB), lambda i: (0, 0)),
            pl.BlockSpec((1, NB), lambda i: (0, 0)),
        ],
        out_shape=[
            jax.ShapeDtypeStruct((AR, AC), jnp.int32),
            jax.ShapeDtypeStruct((1, NB), jnp.int32),
            jax.ShapeDtypeStruct((1, NB), jnp.int32),
        ],
    )(ej)
    pos_flat = pos.reshape(NA)
    p0 = pos_flat[0::2].reshape(1, S)
    p1 = pos_flat[1::2].reshape(1, S)

    xg = _sc_scatter(x2, p0, p1)

    yg = pl.pallas_call(
        _gmm_kernel,
        grid_spec=pltpu.PrefetchScalarGridSpec(
            num_scalar_prefetch=2,
            grid=(NB,),
            in_specs=[
                pl.BlockSpec((BME, D2), lambda i, be_, act_: (i, 0)),
                pl.BlockSpec((1, D, FF),
                             lambda i, be_, act_: (be_[i], 0, 0)),
                pl.BlockSpec((1, D, FF),
                             lambda i, be_, act_: (be_[i], 0, 0)),
                pl.BlockSpec((1, FF, D),
                             lambda i, be_, act_: (be_[i], 0, 0)),
            ],
            out_specs=pl.BlockSpec((BME, D2), lambda i, be_, act_: (i, 0)),
        ),
        out_shape=jax.ShapeDtypeStruct((P, D2), jnp.int32),
        compiler_params=pltpu.CompilerParams(
            dimension_semantics=("arbitrary",)),
    )(be.reshape(NB), act.reshape(NB), xg, wg, wu, wd)

    ab = _sc_gather(yg, jnp.concatenate([p0, p1], axis=1))

    out = pl.pallas_call(
        _final_kernel,
        grid=(NT,),
        in_specs=[
            pl.BlockSpec((BT, D), lambda i: (i, 0)),
            pl.BlockSpec((BT, D2), lambda i: (i, 0)),
            pl.BlockSpec((BT, D2), lambda i: (i + NT, 0)),
            pl.BlockSpec((BT, TOPK), lambda i: (i, 0)),
        ],
        out_specs=pl.BlockSpec((BT, D), lambda i: (i, 0)),
        out_shape=jax.ShapeDtypeStruct((s, d), jnp.float32),
        compiler_params=pltpu.CompilerParams(
            dimension_semantics=("parallel",)),
    )(h2, ab, ab, tw)

    return out.reshape(b, s, d)
